# baseline (speedup 1.0000x reference)
import jax
import jax.numpy as jnp
from jax.experimental import pallas as pl

N = 2048; E = 32768; B = 64
NAF = 16; NBT = 5; SDIM = 256; VDIM = 64; EDIM = 32; NL = 5


def _seg_mean(src, index, num_segments):
    s = jax.ops.segment_sum(src, index, num_segments=num_segments)
    cnt = jax.ops.segment_sum(jnp.ones((src.shape[0],), src.dtype), index, num_segments=num_segments)
    cnt = jnp.maximum(cnt, 1.0)
    return s / cnt.reshape((num_segments,) + (1,) * (src.ndim - 1))


def _bias_add_kernel(x_ref, b_ref, o_ref):
    o_ref[...] = x_ref[...] + b_ref[...]


def _pl_bias_add(x, b):
    return pl.pallas_call(
        _bias_add_kernel,
        out_shape=jax.ShapeDtypeStruct(x.shape, x.dtype),
    )(x, jnp.broadcast_to(b, x.shape))


def kernel(x, t, pos, edge_index, edge_attr, batch, params):
    src = edge_index[0]
    dst = edge_index[1]
    n = x.shape[0]
    s = x @ params['W_atom'] + params['b_atom']
    ta = t @ params['W_ta'] + params['b_ta']
    tb = t @ params['W_tb'] + params['b_tb']
    s = s + (ta[batch] @ params['W_att'] + params['b_att'])
    e = edge_attr @ params['W_bond'] + params['b_bond']
    batch_edge = batch[src]
    e = e + (tb[batch_edge] @ params['W_btt'] + params['b_btt'])
    r = pos[dst] - pos[src]
    pos_norm = jnp.linalg.norm(pos, axis=1, keepdims=True)
    safe = jnp.where(pos_norm == 0.0, 1.0, pos_norm)
    pos_n = jnp.where(pos_norm != 0.0, pos / safe, 0.0)
    a = jnp.sum(pos_n[dst] * pos_n[src], axis=-1)
    d = jnp.sqrt(jnp.clip(jnp.sum(r * r, axis=-1), 1e-6, None))
    r_norm = r / (1.0 + d[:, None])
    v = jnp.zeros((n, 3, VDIM), jnp.float32)
    for l in range(NL):
        m_in = jnp.concatenate([s[src], s[dst], e, d[:, None], a[:, None]], axis=-1)
        m = jax.nn.silu(m_in @ params['W1'][l] + params['b1'][l])
        s = s + (_seg_mean(m, dst, n) @ params['W2'][l] + params['b2'][l])
        e = e + (m @ params['We'][l] + params['be'][l])
        gate = m @ params['Wg'][l]
        gate2 = m @ params['Wg2'][l]
        v_msg = r_norm[:, :, None] * gate[:, None, :] + v[src] * gate2[:, None, :]
        v = v + _seg_mean(v_msg, dst, n)
    s = jax.nn.silu(s @ params['W_sh'] + params['b_sh'])
    coords_pred = jnp.squeeze(jnp.einsum('nid,do->nio', v, params['W_co']), -1)
    atoms_pred = _pl_bias_add(s @ params['W_ao'], params['b_ao'])
    jj, ii = src, dst
    coords_pred = pos + coords_pred
    coords_pred = coords_pred - _seg_mean(coords_pred, batch, B)[batch]
    dd = jnp.sum((coords_pred[ii] - coords_pred[jj]) ** 2, axis=-1, keepdims=True)
    e_dense = jnp.zeros((n, n, e.shape[-1]), jnp.float32).at[src, dst].set(e)
    e_dense = 0.5 * (e_dense + jnp.transpose(e_dense, (1, 0, 2)))
    e_sym = e_dense[src, dst]
    f = s[ii] + s[jj] + (e_sym @ params['W_bm'] + params['b_bm'])
    edge_feat = jnp.concatenate([f, dd], axis=-1)
    bonds_pred = jax.nn.silu(edge_feat @ params['W_b0'] + params['b_b0']) @ params['W_b1'] + params['b_b1']
    return coords_pred, atoms_pred, bonds_pred


# speedup vs baseline: 1.4217x; 1.4217x over previous
import functools

import jax
import jax.numpy as jnp
from jax import lax
from jax.experimental import pallas as pl
from jax.experimental.pallas import tpu as pltpu
from jax.experimental.pallas import tpu_sc as plsc

N = 2048; E = 32768; B = 64
NAF = 16; NBT = 5; SDIM = 256; VDIM = 64; EDIM = 32; NL = 5
V3 = 3 * VDIM

NC = 2
NS = 16
NW = NC * NS
EPW = E // NW
CH = 128
NCHUNK = EPW // CH
TBL = N * N
TPW = TBL // NW
DUMP = TBL

f32 = jnp.float32
i32 = jnp.int32


def _mesh():
    return plsc.VectorSubcoreMesh(core_axis_name="c", subcore_axis_name="s",
                                  num_cores=NC, num_subcores=NS)


_SC_PARAMS = pltpu.CompilerParams(use_tc_tiling_on_sc=False)


def _wid():
    return lax.axis_index("c") * NS + lax.axis_index("s")


@functools.cache
def _sc_gather2():
    @functools.partial(
        pl.kernel,
        out_type=[jax.ShapeDtypeStruct((E, SDIM), f32),
                  jax.ShapeDtypeStruct((E, SDIM), f32)],
        mesh=_mesh(),
        compiler_params=_SC_PARAMS,
        scratch_types=[pltpu.VMEM((CH,), i32), pltpu.VMEM((CH,), i32),
                       pltpu.VMEM((CH, SDIM), f32)],
    )
    def k(p_hbm, q_hbm, src_hbm, dst_hbm, g1_hbm, g2_hbm, isv, idv, rows):
        base0 = _wid() * EPW

        def body(i, _):
            off = base0 + i * CH
            pltpu.sync_copy(src_hbm.at[pl.ds(off, CH)], isv)
            pltpu.sync_copy(dst_hbm.at[pl.ds(off, CH)], idv)
            pltpu.sync_copy(p_hbm.at[isv], rows)
            pltpu.sync_copy(rows, g1_hbm.at[pl.ds(off, CH)])
            pltpu.sync_copy(q_hbm.at[idv], rows)
            pltpu.sync_copy(rows, g2_hbm.at[pl.ds(off, CH)])
            return 0

        lax.fori_loop(0, NCHUNK, body, 0)

    return k


@functools.cache
def _sc_gather3():
    @functools.partial(
        pl.kernel,
        out_type=[jax.ShapeDtypeStruct((E, SDIM), f32),
                  jax.ShapeDtypeStruct((E, SDIM), f32),
                  jax.ShapeDtypeStruct((E, V3), f32)],
        mesh=_mesh(),
        compiler_params=_SC_PARAMS,
        scratch_types=[pltpu.VMEM((CH,), i32), pltpu.VMEM((CH,), i32),
                       pltpu.VMEM((CH, SDIM), f32), pltpu.VMEM((CH, V3), f32)],
    )
    def k(p_hbm, q_hbm, v_hbm, src_hbm, dst_hbm, g1_hbm, g2_hbm, vs_hbm,
          isv, idv, rows, vrows):
        base0 = _wid() * EPW

        def body(i, _):
            off = base0 + i * CH
            pltpu.sync_copy(src_hbm.at[pl.ds(off, CH)], isv)
            pltpu.sync_copy(dst_hbm.at[pl.ds(off, CH)], idv)
            pltpu.sync_copy(p_hbm.at[isv], rows)
            pltpu.sync_copy(rows, g1_hbm.at[pl.ds(off, CH)])
            pltpu.sync_copy(q_hbm.at[idv], rows)
            pltpu.sync_copy(rows, g2_hbm.at[pl.ds(off, CH)])
            pltpu.sync_copy(v_hbm.at[isv], vrows)
            pltpu.sync_copy(vrows, vs_hbm.at[pl.ds(off, CH)])
            return 0

        lax.fori_loop(0, NCHUNK, body, 0)

    return k


@functools.cache
def _sc_pre():
    @functools.partial(
        pl.kernel,
        out_type=[jax.ShapeDtypeStruct((E, 8), f32),
                  jax.ShapeDtypeStruct((E, 8), f32),
                  jax.ShapeDtypeStruct((E, EDIM), f32),
                  jax.ShapeDtypeStruct((NC, N), f32)],
        mesh=_mesh(),
        compiler_params=_SC_PARAMS,
        scratch_types=[pltpu.VMEM((CH,), i32), pltpu.VMEM((CH,), i32),
                       pltpu.VMEM((CH, 8), f32), pltpu.VMEM((CH, EDIM), f32),
                       pltpu.VMEM((CH,), f32),
                       pltpu.VMEM_SHARED((N,), f32)],
    )
    def k(pos_hbm, tb3_hbm, src_hbm, dst_hbm, ones_hbm, zn_hbm,
          ps_hbm, pd_hbm, tbg_hbm, cnt_hbm,
          isv, idv, rows4, rows32, onesv, acc_cnt):
        cid = lax.axis_index("c")
        sid = lax.axis_index("s")
        base0 = (cid * NS + sid) * EPW

        @pl.when(sid == 0)
        def _():
            pltpu.sync_copy(zn_hbm, acc_cnt)

        pltpu.sync_copy(ones_hbm, onesv)
        plsc.subcore_barrier()

        def body(i, _):
            off = base0 + i * CH
            pltpu.sync_copy(src_hbm.at[pl.ds(off, CH)], isv)
            pltpu.sync_copy(dst_hbm.at[pl.ds(off, CH)], idv)
            pltpu.sync_copy(pos_hbm.at[isv], rows4)
            pltpu.sync_copy(rows4, ps_hbm.at[pl.ds(off, CH)])
            pltpu.sync_copy(pos_hbm.at[idv], rows4)
            pltpu.sync_copy(rows4, pd_hbm.at[pl.ds(off, CH)])
            pltpu.sync_copy(tb3_hbm.at[isv], rows32)
            pltpu.sync_copy(rows32, tbg_hbm.at[pl.ds(off, CH)])
            pltpu.sync_copy(onesv, acc_cnt.at[idv], add=True)
            return 0

        lax.fori_loop(0, NCHUNK, body, 0)
        plsc.subcore_barrier()

        @pl.when(sid == 0)
        def _():
            pltpu.sync_copy(acc_cnt, cnt_hbm.at[cid])

    return k


@functools.cache
def _sc_scat2():
    @functools.partial(
        pl.kernel,
        out_type=[jax.ShapeDtypeStruct((NC, N, SDIM), f32),
                  jax.ShapeDtypeStruct((NC, N, V3), f32)],
        mesh=_mesh(),
        compiler_params=_SC_PARAMS,
        scratch_types=[pltpu.VMEM((CH,), i32),
                       pltpu.VMEM((CH, SDIM), f32), pltpu.VMEM((CH, V3), f32),
                       pltpu.VMEM_SHARED((N, SDIM), f32),
                       pltpu.VMEM_SHARED((N, V3), f32)],
    )
    def k(m_hbm, vm_hbm, dst_hbm, zs_hbm, zv_hbm, segm_hbm, segv_hbm,
          idv, mrows, vrows, acc_s, acc_v):
        cid = lax.axis_index("c")
        sid = lax.axis_index("s")
        base0 = (cid * NS + sid) * EPW
        npt = N // NS

        pltpu.sync_copy(zs_hbm.at[pl.ds(sid * npt, npt)], acc_s.at[pl.ds(sid * npt, npt)])
        pltpu.sync_copy(zv_hbm.at[pl.ds(sid * npt, npt)], acc_v.at[pl.ds(sid * npt, npt)])
        plsc.subcore_barrier()

        def body(i, _):
            off = base0 + i * CH
            pltpu.sync_copy(dst_hbm.at[pl.ds(off, CH)], idv)
            pltpu.sync_copy(m_hbm.at[pl.ds(off, CH)], mrows)
            pltpu.sync_copy(vm_hbm.at[pl.ds(off, CH)], vrows)
            pltpu.sync_copy(mrows, acc_s.at[idv], add=True)
            pltpu.sync_copy(vrows, acc_v.at[idv], add=True)
            return 0

        lax.fori_loop(0, NCHUNK, body, 0)
        plsc.subcore_barrier()

        pltpu.sync_copy(acc_s.at[pl.ds(sid * npt, npt)], segm_hbm.at[cid, pl.ds(sid * npt, npt)])
        pltpu.sync_copy(acc_v.at[pl.ds(sid * npt, npt)], segv_hbm.at[cid, pl.ds(sid * npt, npt)])

    return k


@functools.cache
def _sc_sym():
    @functools.partial(
        pl.kernel,
        out_type=[jax.ShapeDtypeStruct((E,), i32),
                  jax.ShapeDtypeStruct((E,), i32),
                  jax.ShapeDtypeStruct((TBL + 8,), i32)],
        mesh=_mesh(),
        compiler_params=_SC_PARAMS,
        scratch_types=[pltpu.VMEM((CH,), i32), pltpu.VMEM((CH,), i32),
                       pltpu.VMEM((CH,), i32), pltpu.VMEM((CH,), i32)],
    )
    def k(key_hbm, rkey_hbm, ids_hbm, neg_hbm, fwd_hbm, rev_hbm, tbl_hbm,
          keyv, idsv, tv, idx2):
        w = _wid()
        base0 = w * EPW

        tb = w * TPW
        def mset(i, _):
            pltpu.sync_copy(neg_hbm, tbl_hbm.at[pl.ds(tb + i * 16384, 16384)])
            return 0
        lax.fori_loop(0, TPW // 16384, mset, 0)

        @pl.when(w == 0)
        def _():
            pltpu.sync_copy(neg_hbm.at[pl.ds(0, 8)], tbl_hbm.at[pl.ds(TBL, 8)])

        plsc.subcore_barrier()

        def scat(i, _):
            off = base0 + i * CH
            pltpu.sync_copy(key_hbm.at[pl.ds(off, CH)], keyv)
            pltpu.sync_copy(ids_hbm.at[pl.ds(off, CH)], idsv)
            pltpu.sync_copy(idsv, tbl_hbm.at[keyv])
            return 0
        lax.fori_loop(0, NCHUNK, scat, 0)
        plsc.subcore_barrier()

        def fix(_p, __):
            def body(i, _):
                off = base0 + i * CH
                pltpu.sync_copy(key_hbm.at[pl.ds(off, CH)], keyv)
                pltpu.sync_copy(ids_hbm.at[pl.ds(off, CH)], idsv)
                pltpu.sync_copy(tbl_hbm.at[keyv], tv)
                def sel(j, _):
                    kv = keyv[pl.ds(j * 16, 16)]
                    iv = idsv[pl.ds(j * 16, 16)]
                    t = tv[pl.ds(j * 16, 16)]
                    idx2[pl.ds(j * 16, 16)] = jnp.where(iv > t, kv, DUMP)
                    return 0
                lax.fori_loop(0, CH // 16, sel, 0)
                pltpu.sync_copy(idsv, tbl_hbm.at[idx2])
                return 0
            lax.fori_loop(0, NCHUNK, body, 0)
            plsc.subcore_barrier()
            return 0
        lax.fori_loop(0, 3, fix, 0)

        def fin(i, _):
            off = base0 + i * CH
            pltpu.sync_copy(key_hbm.at[pl.ds(off, CH)], keyv)
            pltpu.sync_copy(tbl_hbm.at[keyv], tv)
            pltpu.sync_copy(tv, fwd_hbm.at[pl.ds(off, CH)])
            pltpu.sync_copy(rkey_hbm.at[pl.ds(off, CH)], keyv)
            pltpu.sync_copy(tbl_hbm.at[keyv], tv)
            pltpu.sync_copy(tv, rev_hbm.at[pl.ds(off, CH)])
            return 0
        lax.fori_loop(0, NCHUNK, fin, 0)

    return k


@functools.cache
def _sc_head():
    @functools.partial(
        pl.kernel,
        out_type=[jax.ShapeDtypeStruct((E, SDIM), f32),
                  jax.ShapeDtypeStruct((E, SDIM), f32),
                  jax.ShapeDtypeStruct((E, 8), f32),
                  jax.ShapeDtypeStruct((E, 8), f32),
                  jax.ShapeDtypeStruct((E, EDIM), f32),
                  jax.ShapeDtypeStruct((E, EDIM), f32)],
        mesh=_mesh(),
        compiler_params=_SC_PARAMS,
        scratch_types=[pltpu.VMEM((CH,), i32), pltpu.VMEM((CH,), i32),
                       pltpu.VMEM((CH,), i32),
                       pltpu.VMEM((CH, SDIM), f32), pltpu.VMEM((CH, 8), f32),
                       pltpu.VMEM((CH, EDIM), f32)],
    )
    def k(s2_hbm, cp_hbm, eext_hbm, src_hbm, dst_hbm, fwd_hbm, rev_hbm,
          s1o, s2o, cpio, cpjo, efo, ero,
          isv, idv, iwv, rows, rows4, rows32):
        base0 = _wid() * EPW

        def body(i, _):
            off = base0 + i * CH
            pltpu.sync_copy(src_hbm.at[pl.ds(off, CH)], isv)
            pltpu.sync_copy(dst_hbm.at[pl.ds(off, CH)], idv)
            pltpu.sync_copy(s2_hbm.at[idv], rows)
            pltpu.sync_copy(rows, s1o.at[pl.ds(off, CH)])
            pltpu.sync_copy(s2_hbm.at[isv], rows)
            pltpu.sync_copy(rows, s2o.at[pl.ds(off, CH)])
            pltpu.sync_copy(cp_hbm.at[idv], rows4)
            pltpu.sync_copy(rows4, cpio.at[pl.ds(off, CH)])
            pltpu.sync_copy(cp_hbm.at[isv], rows4)
            pltpu.sync_copy(rows4, cpjo.at[pl.ds(off, CH)])
            pltpu.sync_copy(fwd_hbm.at[pl.ds(off, CH)], iwv)
            pltpu.sync_copy(eext_hbm.at[iwv], rows32)
            pltpu.sync_copy(rows32, efo.at[pl.ds(off, CH)])
            pltpu.sync_copy(rev_hbm.at[pl.ds(off, CH)], iwv)
            def sel(j, _):
                rv = iwv[pl.ds(j * 16, 16)]
                iwv[pl.ds(j * 16, 16)] = jnp.where(rv < 0, E, rv)
                return 0
            lax.fori_loop(0, CH // 16, sel, 0)
            pltpu.sync_copy(eext_hbm.at[iwv], rows32)
            pltpu.sync_copy(rows32, ero.at[pl.ds(off, CH)])
            return 0

        lax.fori_loop(0, NCHUNK, body, 0)

    return k


def _silu(x):
    return x * jax.nn.sigmoid(x)


@functools.cache
def _tc_pre_node():
    def body(x_ref, t_ref, batch_ref, cnt_ref, W_atom, b_atom, W_ta, b_ta,
             W_att, b_att, W_tb, b_tb, W_btt, b_btt, W1a, W1b,
             s_ref, p_ref, q_ref, tb3_ref, icnt_ref):
        t = t_ref[...]
        ta2 = jnp.dot(t * W_ta[...] + b_ta[...], W_att[...],
                      preferred_element_type=f32) + b_att[...]
        tb2 = (t * W_tb[...] + b_tb[...]) @ W_btt[...] + b_btt[...]
        oh = (batch_ref[...] == lax.broadcasted_iota(i32, (1, B), 1)).astype(f32)
        s0 = jnp.dot(x_ref[...], W_atom[...], preferred_element_type=f32) \
            + b_atom[...] + jnp.dot(oh, ta2, preferred_element_type=f32)
        s_ref[...] = s0
        tb3_ref[...] = jnp.dot(oh, tb2, preferred_element_type=f32)
        p_ref[...] = jnp.dot(s0, W1a[...], preferred_element_type=f32)
        q_ref[...] = jnp.dot(s0, W1b[...], preferred_element_type=f32)
        cnt = cnt_ref[0, :] + cnt_ref[1, :]
        icnt_ref[...] = (1.0 / jnp.maximum(cnt, 1.0)).reshape(N, 1)

    return pl.pallas_call(
        body,
        out_shape=[jax.ShapeDtypeStruct((N, SDIM), f32),
                   jax.ShapeDtypeStruct((N, SDIM), f32),
                   jax.ShapeDtypeStruct((N, SDIM), f32),
                   jax.ShapeDtypeStruct((N, EDIM), f32),
                   jax.ShapeDtypeStruct((N, 1), f32)],
    )


BE = 1024


@functools.cache
def _tc_pre_edge():
    def body(ea_ref, tbg_ref, ps_ref, pd_ref, src_ref, dst_ref, W_bond, b_bond,
             e_ref, d_ref, a_ref, rn_ref, key_ref, rkey_ref):
        ps = ps_ref[...]
        pd = pd_ref[...]
        e_ref[...] = jnp.dot(ea_ref[...], W_bond[...], preferred_element_type=f32) \
            + b_bond[...] + tbg_ref[...]
        r = pd - ps
        rr = jnp.sum(r * r, axis=-1, keepdims=True)
        d = jnp.sqrt(jnp.clip(rr, 1e-6, None))
        d_ref[...] = d
        ns = jnp.sqrt(jnp.sum(ps * ps, axis=-1, keepdims=True))
        nd = jnp.sqrt(jnp.sum(pd * pd, axis=-1, keepdims=True))
        pns = jnp.where(ns != 0.0, ps / jnp.where(ns == 0.0, 1.0, ns), 0.0)
        pnd = jnp.where(nd != 0.0, pd / jnp.where(nd == 0.0, 1.0, nd), 0.0)
        a_ref[...] = jnp.sum(pns * pnd, axis=-1, keepdims=True)
        rn_ref[...] = (r / (1.0 + d))[:, 0:4]
        key_ref[...] = src_ref[...] * N + dst_ref[...]
        rkey_ref[...] = dst_ref[...] * N + src_ref[...]

    grid = (E // BE,)
    eb = lambda w: pl.BlockSpec((BE, w), lambda i: (i, 0))
    wb = lambda s: pl.BlockSpec(s, lambda i: (0,) * len(s))
    return pl.pallas_call(
        body,
        grid=grid,
        in_specs=[eb(NBT), eb(EDIM), eb(8), eb(8), eb(1), eb(1),
                  wb((NBT, EDIM)), wb((1, EDIM))],
        out_specs=[eb(EDIM), eb(1), eb(1), eb(4), eb(1), eb(1)],
        out_shape=[jax.ShapeDtypeStruct((E, EDIM), f32),
                   jax.ShapeDtypeStruct((E, 1), f32),
                   jax.ShapeDtypeStruct((E, 1), f32),
                   jax.ShapeDtypeStruct((E, 4), f32),
                   jax.ShapeDtypeStruct((E, 1), i32),
                   jax.ShapeDtypeStruct((E, 1), i32)],
    )


@functools.cache
def _tc_layer_node():
    def body(s_ref, v_ref, sg0_ref, sg1_ref, vp0_ref, vp1_ref, icnt_ref,
             W2, b2, W1a, W1b, s_new_ref, v_new_ref, p_ref, q_ref):
        icnt = icnt_ref[...]
        seg = (sg0_ref[...] + sg1_ref[...]) * icnt
        s_new = s_ref[...] + jnp.dot(seg, W2[...], preferred_element_type=f32) + b2[...]
        s_new_ref[...] = s_new
        v_new_ref[...] = v_ref[...] + (vp0_ref[...] + vp1_ref[...]) * icnt
        p_ref[...] = jnp.dot(s_new, W1a[...], preferred_element_type=f32)
        q_ref[...] = jnp.dot(s_new, W1b[...], preferred_element_type=f32)

    return pl.pallas_call(
        body,
        out_shape=[jax.ShapeDtypeStruct((N, SDIM), f32),
                   jax.ShapeDtypeStruct((N, V3), f32),
                   jax.ShapeDtypeStruct((N, SDIM), f32),
                   jax.ShapeDtypeStruct((N, SDIM), f32)],
    )


def _tc_edge_msg_body(first, g1_ref, g2_ref, vs_ref, e_ref, d_ref, a_ref, rn_ref,
                      W1c, wd, wa, b1, We, be_, Wg, Wg2,
                      m_ref, enew_ref, vm_ref):
    pre = g1_ref[...] + g2_ref[...] \
        + jnp.dot(e_ref[...], W1c[...], preferred_element_type=f32) \
        + d_ref[...] * wd[...] + a_ref[...] * wa[...] + b1[...]
    m = _silu(pre)
    m_ref[...] = m
    enew_ref[...] = e_ref[...] + jnp.dot(m, We[...], preferred_element_type=f32) + be_[...]
    g = jnp.dot(m, Wg[...], preferred_element_type=f32)
    rn = rn_ref[...]
    vm = jnp.concatenate([rn[:, 0:1] * g, rn[:, 1:2] * g, rn[:, 2:3] * g], axis=-1)
    if not first:
        g2g = jnp.dot(m, Wg2[...], preferred_element_type=f32)
        vm = vm + vs_ref[...] * jnp.concatenate([g2g, g2g, g2g], axis=-1)
    vm_ref[...] = vm


@functools.cache
def _tc_edge_msg(first):
    grid = (E // BE,)
    eb = lambda w: pl.BlockSpec((BE, w), lambda i: (i, 0))
    wb = lambda s: pl.BlockSpec(s, lambda i: (0,) * len(s))
    in_specs = [eb(SDIM), eb(SDIM)]
    if not first:
        in_specs.append(eb(V3))
    in_specs += [eb(EDIM), eb(1), eb(1), eb(4),
                 wb((EDIM, SDIM)), wb((1, SDIM)), wb((1, SDIM)), wb((1, SDIM)),
                 wb((SDIM, EDIM)), wb((1, EDIM)),
                 wb((SDIM, VDIM)), wb((SDIM, VDIM))]

    def body(*refs):
        if first:
            g1, g2, e, d, a, rn, W1c, wd, wa, b1, We, be_, Wg, Wg2, m, en, vm = refs
            _tc_edge_msg_body(True, g1, g2, None, e, d, a, rn,
                              W1c, wd, wa, b1, We, be_, Wg, Wg2, m, en, vm)
        else:
            g1, g2, vs, e, d, a, rn, W1c, wd, wa, b1, We, be_, Wg, Wg2, m, en, vm = refs
            _tc_edge_msg_body(False, g1, g2, vs, e, d, a, rn,
                              W1c, wd, wa, b1, We, be_, Wg, Wg2, m, en, vm)

    return pl.pallas_call(
        body,
        grid=grid,
        in_specs=in_specs,
        out_specs=[eb(SDIM), eb(EDIM), eb(V3)],
        out_shape=[jax.ShapeDtypeStruct((E, SDIM), f32),
                   jax.ShapeDtypeStruct((E, EDIM), f32),
                   jax.ShapeDtypeStruct((E, V3), f32)],
    )


@functools.cache
def _tc_head_node():
    def body(s_ref, v_ref, sg0_ref, sg1_ref, vp0_ref, vp1_ref, icnt_ref,
             pos_ref, batch_ref, W2, b2, W_sh, b_sh, W_ao, b_ao, W_co,
             W_b0a, W_bm, b_bm, b_b0,
             atoms_ref, cp_ref, s2_ref, wcomb_ref, bias0_ref):
        icnt = icnt_ref[...]
        seg = (sg0_ref[...] + sg1_ref[...]) * icnt
        s5 = s_ref[...] + jnp.dot(seg, W2[...], preferred_element_type=f32) + b2[...]
        v5 = v_ref[...] + (vp0_ref[...] + vp1_ref[...]) * icnt
        sh = _silu(jnp.dot(s5, W_sh[...], preferred_element_type=f32) + b_sh[...])
        atoms_ref[...] = jnp.dot(sh, W_ao[...], preferred_element_type=f32) + b_ao[...]
        wco = W_co[...]
        c0 = jnp.sum(v5[:, 0:VDIM] * wco, axis=-1, keepdims=True)
        c1 = jnp.sum(v5[:, VDIM:2 * VDIM] * wco, axis=-1, keepdims=True)
        c2 = jnp.sum(v5[:, 2 * VDIM:] * wco, axis=-1, keepdims=True)
        cp0 = pos_ref[...] + jnp.concatenate(
            [c0, c1, c2, jnp.zeros((N, 5), f32)], axis=-1)
        oh = (batch_ref[...] == lax.broadcasted_iota(i32, (1, B), 1)).astype(f32)
        cp0e = jnp.concatenate([cp0, jnp.ones((N, 1), f32)], axis=-1)
        sums = lax.dot_general(oh, cp0e, (((0,), (0,)), ((), ())),
                               preferred_element_type=f32)
        means = sums[:, 0:8] / jnp.maximum(sums[:, 8:9], 1.0)
        cp_ref[...] = cp0 - jnp.dot(oh, means, preferred_element_type=f32)
        s2_ref[...] = jnp.dot(sh, W_b0a[...], preferred_element_type=f32)
        wcomb_ref[...] = jnp.dot(W_bm[...], W_b0a[...], preferred_element_type=f32)
        bias0_ref[...] = jnp.dot(b_bm[...], W_b0a[...],
                                 preferred_element_type=f32) + b_b0[...]

    return pl.pallas_call(
        body,
        out_shape=[jax.ShapeDtypeStruct((N, NAF), f32),
                   jax.ShapeDtypeStruct((N, 8), f32),
                   jax.ShapeDtypeStruct((N, SDIM), f32),
                   jax.ShapeDtypeStruct((EDIM, SDIM), f32),
                   jax.ShapeDtypeStruct((1, SDIM), f32)],
    )


@functools.cache
def _tc_head_edge():
    def body(s1_ref, s2_ref, ef_ref, er_ref, cpi_ref, cpj_ref,
             wcomb, bias0, wdd, W_b1, b_b1, bonds_ref):
        diff = cpi_ref[...] - cpj_ref[...]
        dd = jnp.sum(diff * diff, axis=-1, keepdims=True)
        es = 0.5 * (ef_ref[...] + er_ref[...])
        pre = s1_ref[...] + s2_ref[...] \
            + jnp.dot(es, wcomb[...], preferred_element_type=f32) \
            + dd * wdd[...] + bias0[...]
        h = _silu(pre)
        bonds_ref[...] = jnp.dot(h, W_b1[...], preferred_element_type=f32) + b_b1[...]

    grid = (E // BE,)
    eb = lambda w: pl.BlockSpec((BE, w), lambda i: (i, 0))
    wb = lambda s: pl.BlockSpec(s, lambda i: (0,) * len(s))
    return pl.pallas_call(
        body,
        grid=grid,
        in_specs=[eb(SDIM), eb(SDIM), eb(EDIM), eb(EDIM), eb(8), eb(8),
                  wb((EDIM, SDIM)), wb((1, SDIM)), wb((1, SDIM)),
                  wb((SDIM, NBT)), wb((1, NBT))],
        out_specs=[eb(NBT)],
        out_shape=[jax.ShapeDtypeStruct((E, NBT), f32)],
    )


def kernel(x, t, pos, edge_index, edge_attr, batch, params):
    src = edge_index[0]
    dst = edge_index[1]
    batch2d = batch.reshape(N, 1)
    pos4 = jnp.concatenate([pos, jnp.zeros((N, 5), f32)], axis=-1)
    src2d = src.reshape(E, 1)
    dst2d = dst.reshape(E, 1)

    W1 = params['W1']
    W1a = [W1[l, :SDIM] for l in range(NL)]
    W1b = [W1[l, SDIM:2 * SDIM] for l in range(NL)]
    W1c = [W1[l, 2 * SDIM:2 * SDIM + EDIM] for l in range(NL)]
    wd = [W1[l, 2 * SDIM + EDIM].reshape(1, SDIM) for l in range(NL)]
    wa = [W1[l, 2 * SDIM + EDIM + 1].reshape(1, SDIM) for l in range(NL)]
    b1 = [params['b1'][l].reshape(1, SDIM) for l in range(NL)]

    def tb3_body(t_ref, batch_ref, W_tb, b_tb, W_btt, b_btt, tb3_ref):
        tb2 = (t_ref[...] * W_tb[...] + b_tb[...]) @ W_btt[...] + b_btt[...]
        oh = (batch_ref[...] == lax.broadcasted_iota(i32, (1, B), 1)).astype(f32)
        tb3_ref[...] = jnp.dot(oh, tb2, preferred_element_type=f32)

    tb3 = pl.pallas_call(
        tb3_body, out_shape=jax.ShapeDtypeStruct((N, EDIM), f32),
    )(t, batch2d, params['W_tb'], params['b_tb'], params['W_btt'], params['b_btt'])

    ones_ch = jnp.ones((CH,), f32)
    zeros_n = jnp.zeros((N,), f32)
    ps4, pd4, tbg, cnt2 = _sc_pre()(pos4, tb3, src, dst, ones_ch, zeros_n)

    s0, P, Q, _tb3_unused, inv_cnt = _tc_pre_node()(
        x, t, batch2d, cnt2, params['W_atom'], params['b_atom'],
        params['W_ta'], params['b_ta'], params['W_att'], params['b_att'],
        params['W_tb'], params['b_tb'], params['W_btt'], params['b_btt'],
        W1a[0], W1b[0])

    e0, d_e, a_e, rn4, key2d, rkey2d = _tc_pre_edge()(
        edge_attr, tbg, ps4, pd4, src2d, dst2d,
        params['W_bond'], params['b_bond'].reshape(1, EDIM))

    zs = jnp.zeros((N, SDIM), f32)
    zv = jnp.zeros((N, V3), f32)

    s, e, v = s0, e0, None
    segm = segv = None
    for l in range(NL):
        if l == 0:
            G1, G2 = _sc_gather2()(P, Q, src, dst)
            m, e, vm = _tc_edge_msg(True)(
                G1, G2, e, d_e, a_e, rn4,
                W1c[0], wd[0], wa[0], b1[0],
                params['We'][0], params['be'][0].reshape(1, EDIM),
                params['Wg'][0], params['Wg2'][0])
        else:
            s, v, P, Q = _tc_layer_node()(
                s, v if v is not None else jnp.zeros((N, V3), f32),
                segm[0], segm[1], segv[0], segv[1], inv_cnt,
                params['W2'][l - 1], params['b2'][l - 1].reshape(1, SDIM),
                W1a[l], W1b[l])
            G1, G2, Vs = _sc_gather3()(P, Q, v, src, dst)
            m, e, vm = _tc_edge_msg(False)(
                G1, G2, Vs, e, d_e, a_e, rn4,
                W1c[l], wd[l], wa[l], b1[l],
                params['We'][l], params['be'][l].reshape(1, EDIM),
                params['Wg'][l], params['Wg2'][l])
        segm, segv = _sc_scat2()(m, vm, dst, zs, zv)
        if l == 0:
            v = jnp.zeros((N, V3), f32)

    atoms_pred, cp4, s2, wcomb, bias0 = _tc_head_node()(
        s, v, segm[0], segm[1], segv[0], segv[1], inv_cnt, pos4, batch2d,
        params['W2'][NL - 1], params['b2'][NL - 1].reshape(1, SDIM),
        params['W_sh'], params['b_sh'].reshape(1, SDIM),
        params['W_ao'], params['b_ao'].reshape(1, NAF),
        params['W_co'].reshape(1, VDIM),
        params['W_b0'][:SDIM], params['W_bm'],
        params['b_bm'].reshape(1, SDIM), params['b_b0'].reshape(1, SDIM))

    key = key2d.reshape(E)
    rkey = rkey2d.reshape(E)
    ids = jnp.arange(E, dtype=i32)
    neg1 = jnp.full((16384,), -1, i32)
    fwd, rev, _tbl = _sc_sym()(key, rkey, ids, neg1)

    e_ext = jnp.concatenate([e, jnp.zeros((8, EDIM), f32)], axis=0)
    S1, S2, cpi, cpj, ef, er = _sc_head()(s2, cp4, e_ext, src, dst, fwd, rev)

    bonds_pred, = _tc_head_edge()(
        S1, S2, ef, er, cpi, cpj, wcomb, bias0,
        params['W_b0'][SDIM].reshape(1, SDIM),
        params['W_b1'], params['b_b1'].reshape(1, NBT))

    coords_pred = cp4[:, :3]
    return coords_pred, atoms_pred, bonds_pred


# speedup vs baseline: 1.5127x; 1.0640x over previous
import functools

import jax
import jax.numpy as jnp
from jax import lax
from jax.experimental import pallas as pl
from jax.experimental.pallas import tpu as pltpu
from jax.experimental.pallas import tpu_sc as plsc

N = 2048; E = 32768; B = 64
NAF = 16; NBT = 5; SDIM = 256; VDIM = 64; EDIM = 32; NL = 5
V3 = 3 * VDIM

NC = 2
NS = 16
NW = NC * NS
EPW = E // NW
CH = 128
NCHUNK = EPW // CH
TBL = N * N
TPW = TBL // NW
DUMP = TBL

f32 = jnp.float32
i32 = jnp.int32


def _mesh():
    return plsc.VectorSubcoreMesh(core_axis_name="c", subcore_axis_name="s",
                                  num_cores=NC, num_subcores=NS)


_SC_PARAMS = pltpu.CompilerParams(use_tc_tiling_on_sc=False)


def _wid():
    return lax.axis_index("c") * NS + lax.axis_index("s")


@functools.cache
def _sc_gather2():
    @functools.partial(
        pl.kernel,
        out_type=[jax.ShapeDtypeStruct((E, SDIM), f32),
                  jax.ShapeDtypeStruct((E, SDIM), f32)],
        mesh=_mesh(),
        compiler_params=_SC_PARAMS,
        scratch_types=[pltpu.VMEM((CH,), i32), pltpu.VMEM((CH,), i32),
                       pltpu.VMEM((CH, SDIM), f32)],
    )
    def k(p_hbm, q_hbm, src_hbm, dst_hbm, g1_hbm, g2_hbm, isv, idv, rows):
        base0 = _wid() * EPW

        def body(i, _):
            off = base0 + i * CH
            pltpu.sync_copy(src_hbm.at[pl.ds(off, CH)], isv)
            pltpu.sync_copy(dst_hbm.at[pl.ds(off, CH)], idv)
            pltpu.sync_copy(p_hbm.at[isv], rows)
            pltpu.sync_copy(rows, g1_hbm.at[pl.ds(off, CH)])
            pltpu.sync_copy(q_hbm.at[idv], rows)
            pltpu.sync_copy(rows, g2_hbm.at[pl.ds(off, CH)])
            return 0

        lax.fori_loop(0, NCHUNK, body, 0)

    return k


@functools.cache
def _sc_gather3():
    @functools.partial(
        pl.kernel,
        out_type=[jax.ShapeDtypeStruct((E, SDIM), f32),
                  jax.ShapeDtypeStruct((E, SDIM), f32),
                  jax.ShapeDtypeStruct((E, V3), f32)],
        mesh=_mesh(),
        compiler_params=_SC_PARAMS,
        scratch_types=[pltpu.VMEM((CH,), i32), pltpu.VMEM((CH,), i32),
                       pltpu.VMEM((CH, SDIM), f32), pltpu.VMEM((CH, V3), f32)],
    )
    def k(p_hbm, q_hbm, v_hbm, src_hbm, dst_hbm, g1_hbm, g2_hbm, vs_hbm,
          isv, idv, rows, vrows):
        base0 = _wid() * EPW

        def body(i, _):
            off = base0 + i * CH
            pltpu.sync_copy(src_hbm.at[pl.ds(off, CH)], isv)
            pltpu.sync_copy(dst_hbm.at[pl.ds(off, CH)], idv)
            pltpu.sync_copy(p_hbm.at[isv], rows)
            pltpu.sync_copy(rows, g1_hbm.at[pl.ds(off, CH)])
            pltpu.sync_copy(q_hbm.at[idv], rows)
            pltpu.sync_copy(rows, g2_hbm.at[pl.ds(off, CH)])
            pltpu.sync_copy(v_hbm.at[isv], vrows)
            pltpu.sync_copy(vrows, vs_hbm.at[pl.ds(off, CH)])
            return 0

        lax.fori_loop(0, NCHUNK, body, 0)

    return k


@functools.cache
def _sc_pre():
    @functools.partial(
        pl.kernel,
        out_type=[jax.ShapeDtypeStruct((E, 8), f32),
                  jax.ShapeDtypeStruct((E, 8), f32),
                  jax.ShapeDtypeStruct((E, EDIM), f32),
                  jax.ShapeDtypeStruct((NC, N), f32)],
        mesh=_mesh(),
        compiler_params=_SC_PARAMS,
        scratch_types=[pltpu.VMEM((CH,), i32), pltpu.VMEM((CH,), i32),
                       pltpu.VMEM((CH, 8), f32), pltpu.VMEM((CH, EDIM), f32),
                       pltpu.VMEM((CH,), f32),
                       pltpu.VMEM_SHARED((N,), f32)],
    )
    def k(pos_hbm, tb3_hbm, src_hbm, dst_hbm, ones_hbm, zn_hbm,
          ps_hbm, pd_hbm, tbg_hbm, cnt_hbm,
          isv, idv, rows4, rows32, onesv, acc_cnt):
        cid = lax.axis_index("c")
        sid = lax.axis_index("s")
        base0 = (cid * NS + sid) * EPW

        @pl.when(sid == 0)
        def _():
            pltpu.sync_copy(zn_hbm, acc_cnt)

        pltpu.sync_copy(ones_hbm, onesv)
        plsc.subcore_barrier()

        def body(i, _):
            off = base0 + i * CH
            pltpu.sync_copy(src_hbm.at[pl.ds(off, CH)], isv)
            pltpu.sync_copy(dst_hbm.at[pl.ds(off, CH)], idv)
            pltpu.sync_copy(pos_hbm.at[isv], rows4)
            pltpu.sync_copy(rows4, ps_hbm.at[pl.ds(off, CH)])
            pltpu.sync_copy(pos_hbm.at[idv], rows4)
            pltpu.sync_copy(rows4, pd_hbm.at[pl.ds(off, CH)])
            pltpu.sync_copy(tb3_hbm.at[isv], rows32)
            pltpu.sync_copy(rows32, tbg_hbm.at[pl.ds(off, CH)])
            pltpu.sync_copy(onesv, acc_cnt.at[idv], add=True)
            return 0

        lax.fori_loop(0, NCHUNK, body, 0)
        plsc.subcore_barrier()

        @pl.when(sid == 0)
        def _():
            pltpu.sync_copy(acc_cnt, cnt_hbm.at[cid])

    return k


@functools.cache
def _sc_scat2():
    @functools.partial(
        pl.kernel,
        out_type=[jax.ShapeDtypeStruct((NC, N, SDIM), f32),
                  jax.ShapeDtypeStruct((NC, N, V3), f32)],
        mesh=_mesh(),
        compiler_params=_SC_PARAMS,
        scratch_types=[pltpu.VMEM((CH,), i32),
                       pltpu.VMEM((CH, SDIM), f32), pltpu.VMEM((CH, V3), f32),
                       pltpu.VMEM_SHARED((N, SDIM), f32),
                       pltpu.VMEM_SHARED((N, V3), f32)],
    )
    def k(m_hbm, vm_hbm, dst_hbm, zs_hbm, zv_hbm, segm_hbm, segv_hbm,
          idv, mrows, vrows, acc_s, acc_v):
        cid = lax.axis_index("c")
        sid = lax.axis_index("s")
        base0 = (cid * NS + sid) * EPW
        npt = N // NS

        pltpu.sync_copy(zs_hbm.at[pl.ds(sid * npt, npt)], acc_s.at[pl.ds(sid * npt, npt)])
        pltpu.sync_copy(zv_hbm.at[pl.ds(sid * npt, npt)], acc_v.at[pl.ds(sid * npt, npt)])
        plsc.subcore_barrier()

        def body(i, _):
            off = base0 + i * CH
            pltpu.sync_copy(dst_hbm.at[pl.ds(off, CH)], idv)
            pltpu.sync_copy(m_hbm.at[pl.ds(off, CH)], mrows)
            pltpu.sync_copy(vm_hbm.at[pl.ds(off, CH)], vrows)
            pltpu.sync_copy(mrows, acc_s.at[idv], add=True)
            pltpu.sync_copy(vrows, acc_v.at[idv], add=True)
            return 0

        lax.fori_loop(0, NCHUNK, body, 0)
        plsc.subcore_barrier()

        pltpu.sync_copy(acc_s.at[pl.ds(sid * npt, npt)], segm_hbm.at[cid, pl.ds(sid * npt, npt)])
        pltpu.sync_copy(acc_v.at[pl.ds(sid * npt, npt)], segv_hbm.at[cid, pl.ds(sid * npt, npt)])

    return k


@functools.cache
def _sc_sym():
    @functools.partial(
        pl.kernel,
        out_type=[jax.ShapeDtypeStruct((E,), i32),
                  jax.ShapeDtypeStruct((E,), i32),
                  jax.ShapeDtypeStruct((TBL + 8,), i32)],
        mesh=_mesh(),
        compiler_params=_SC_PARAMS,
        scratch_types=[pltpu.VMEM((CH,), i32), pltpu.VMEM((CH,), i32),
                       pltpu.VMEM((CH,), i32), pltpu.VMEM((CH,), i32),
                       pltpu.VMEM((16384,), i32)],
    )
    def k(key_hbm, rkey_hbm, ids_hbm, neg_hbm, fwd_hbm, rev_hbm, tbl_hbm,
          keyv, idsv, tv, idx2, negv):
        w = _wid()
        base0 = w * EPW

        tb = w * TPW
        pltpu.sync_copy(neg_hbm, negv)
        def mset(i, _):
            pltpu.sync_copy(negv, tbl_hbm.at[pl.ds(tb + i * 16384, 16384)])
            return 0
        lax.fori_loop(0, TPW // 16384, mset, 0)

        @pl.when(w == 0)
        def _():
            pltpu.sync_copy(negv.at[pl.ds(0, 8)], tbl_hbm.at[pl.ds(TBL, 8)])

        plsc.subcore_barrier()

        def scat(i, _):
            off = base0 + i * CH
            pltpu.sync_copy(key_hbm.at[pl.ds(off, CH)], keyv)
            pltpu.sync_copy(ids_hbm.at[pl.ds(off, CH)], idsv)
            pltpu.sync_copy(idsv, tbl_hbm.at[keyv])
            return 0
        lax.fori_loop(0, NCHUNK, scat, 0)
        plsc.subcore_barrier()

        def fix(_p, __):
            def body(i, _):
                off = base0 + i * CH
                pltpu.sync_copy(key_hbm.at[pl.ds(off, CH)], keyv)
                pltpu.sync_copy(ids_hbm.at[pl.ds(off, CH)], idsv)
                pltpu.sync_copy(tbl_hbm.at[keyv], tv)
                def sel(j, _):
                    kv = keyv[pl.ds(j * 16, 16)]
                    iv = idsv[pl.ds(j * 16, 16)]
                    t = tv[pl.ds(j * 16, 16)]
                    idx2[pl.ds(j * 16, 16)] = jnp.where(iv > t, kv, DUMP)
                    return 0
                lax.fori_loop(0, CH // 16, sel, 0)
                pltpu.sync_copy(idsv, tbl_hbm.at[idx2])
                return 0
            lax.fori_loop(0, NCHUNK, body, 0)
            plsc.subcore_barrier()
            return 0
        lax.fori_loop(0, 3, fix, 0)

        def fin(i, _):
            off = base0 + i * CH
            pltpu.sync_copy(key_hbm.at[pl.ds(off, CH)], keyv)
            pltpu.sync_copy(tbl_hbm.at[keyv], tv)
            pltpu.sync_copy(tv, fwd_hbm.at[pl.ds(off, CH)])
            pltpu.sync_copy(rkey_hbm.at[pl.ds(off, CH)], keyv)
            pltpu.sync_copy(tbl_hbm.at[keyv], tv)
            pltpu.sync_copy(tv, rev_hbm.at[pl.ds(off, CH)])
            return 0
        lax.fori_loop(0, NCHUNK, fin, 0)

    return k


@functools.cache
def _sc_head():
    @functools.partial(
        pl.kernel,
        out_type=[jax.ShapeDtypeStruct((E, SDIM), f32),
                  jax.ShapeDtypeStruct((E, SDIM), f32),
                  jax.ShapeDtypeStruct((E, 8), f32),
                  jax.ShapeDtypeStruct((E, 8), f32),
                  jax.ShapeDtypeStruct((E, EDIM), f32),
                  jax.ShapeDtypeStruct((E, EDIM), f32)],
        mesh=_mesh(),
        compiler_params=_SC_PARAMS,
        scratch_types=[pltpu.VMEM((CH,), i32), pltpu.VMEM((CH,), i32),
                       pltpu.VMEM((CH,), i32),
                       pltpu.VMEM((CH, SDIM), f32), pltpu.VMEM((CH, 8), f32),
                       pltpu.VMEM((CH, EDIM), f32)],
    )
    def k(s2_hbm, cp_hbm, eext_hbm, src_hbm, dst_hbm, fwd_hbm, rev_hbm,
          s1o, s2o, cpio, cpjo, efo, ero,
          isv, idv, iwv, rows, rows4, rows32):
        base0 = _wid() * EPW

        def body(i, _):
            off = base0 + i * CH
            pltpu.sync_copy(src_hbm.at[pl.ds(off, CH)], isv)
            pltpu.sync_copy(dst_hbm.at[pl.ds(off, CH)], idv)
            pltpu.sync_copy(s2_hbm.at[idv], rows)
            pltpu.sync_copy(rows, s1o.at[pl.ds(off, CH)])
            pltpu.sync_copy(s2_hbm.at[isv], rows)
            pltpu.sync_copy(rows, s2o.at[pl.ds(off, CH)])
            pltpu.sync_copy(cp_hbm.at[idv], rows4)
            pltpu.sync_copy(rows4, cpio.at[pl.ds(off, CH)])
            pltpu.sync_copy(cp_hbm.at[isv], rows4)
            pltpu.sync_copy(rows4, cpjo.at[pl.ds(off, CH)])
            pltpu.sync_copy(fwd_hbm.at[pl.ds(off, CH)], iwv)
            pltpu.sync_copy(eext_hbm.at[iwv], rows32)
            pltpu.sync_copy(rows32, efo.at[pl.ds(off, CH)])
            pltpu.sync_copy(rev_hbm.at[pl.ds(off, CH)], iwv)
            def sel(j, _):
                rv = iwv[pl.ds(j * 16, 16)]
                iwv[pl.ds(j * 16, 16)] = jnp.where(rv < 0, E, rv)
                return 0
            lax.fori_loop(0, CH // 16, sel, 0)
            pltpu.sync_copy(eext_hbm.at[iwv], rows32)
            pltpu.sync_copy(rows32, ero.at[pl.ds(off, CH)])
            return 0

        lax.fori_loop(0, NCHUNK, body, 0)

    return k


def _silu(x):
    return x * jax.nn.sigmoid(x)


@functools.cache
def _tc_pre_node():
    def body(x_ref, t_ref, batch_ref, cnt_ref, W_atom, b_atom, W_ta, b_ta,
             W_att, b_att, W_tb, b_tb, W_btt, b_btt, W1a, W1b,
             s_ref, p_ref, q_ref, tb3_ref, icnt_ref):
        t = t_ref[...]
        ta2 = jnp.dot(t * W_ta[...] + b_ta[...], W_att[...],
                      preferred_element_type=f32) + b_att[...]
        tb2 = (t * W_tb[...] + b_tb[...]) @ W_btt[...] + b_btt[...]
        oh = (batch_ref[...] == lax.broadcasted_iota(i32, (1, B), 1)).astype(f32)
        s0 = jnp.dot(x_ref[...], W_atom[...], preferred_element_type=f32) \
            + b_atom[...] + jnp.dot(oh, ta2, preferred_element_type=f32)
        s_ref[...] = s0
        tb3_ref[...] = jnp.dot(oh, tb2, preferred_element_type=f32)
        p_ref[...] = jnp.dot(s0, W1a[...], preferred_element_type=f32)
        q_ref[...] = jnp.dot(s0, W1b[...], preferred_element_type=f32)
        cnt = cnt_ref[0, :] + cnt_ref[1, :]
        icnt_ref[...] = (1.0 / jnp.maximum(cnt, 1.0)).reshape(N, 1)

    return pl.pallas_call(
        body,
        out_shape=[jax.ShapeDtypeStruct((N, SDIM), f32),
                   jax.ShapeDtypeStruct((N, SDIM), f32),
                   jax.ShapeDtypeStruct((N, SDIM), f32),
                   jax.ShapeDtypeStruct((N, EDIM), f32),
                   jax.ShapeDtypeStruct((N, 1), f32)],
    )


BE = 1024


@functools.cache
def _tc_pre_edge():
    def body(ea_ref, tbg_ref, ps_ref, pd_ref, src_ref, dst_ref, W_bond, b_bond,
             e_ref, d_ref, a_ref, rn_ref, key_ref, rkey_ref):
        ps = ps_ref[...]
        pd = pd_ref[...]
        e_ref[...] = jnp.dot(ea_ref[...], W_bond[...], preferred_element_type=f32) \
            + b_bond[...] + tbg_ref[...]
        r = pd - ps
        rr = jnp.sum(r * r, axis=-1, keepdims=True)
        d = jnp.sqrt(jnp.clip(rr, 1e-6, None))
        d_ref[...] = d
        ns = jnp.sqrt(jnp.sum(ps * ps, axis=-1, keepdims=True))
        nd = jnp.sqrt(jnp.sum(pd * pd, axis=-1, keepdims=True))
        pns = jnp.where(ns != 0.0, ps / jnp.where(ns == 0.0, 1.0, ns), 0.0)
        pnd = jnp.where(nd != 0.0, pd / jnp.where(nd == 0.0, 1.0, nd), 0.0)
        a_ref[...] = jnp.sum(pns * pnd, axis=-1, keepdims=True)
        rn_ref[...] = (r / (1.0 + d))[:, 0:4]
        key_ref[...] = src_ref[...] * N + dst_ref[...]
        rkey_ref[...] = dst_ref[...] * N + src_ref[...]

    grid = (E // BE,)
    eb = lambda w: pl.BlockSpec((BE, w), lambda i: (i, 0))
    wb = lambda s: pl.BlockSpec(s, lambda i: (0,) * len(s))
    return pl.pallas_call(
        body,
        grid=grid,
        in_specs=[eb(NBT), eb(EDIM), eb(8), eb(8), eb(1), eb(1),
                  wb((NBT, EDIM)), wb((1, EDIM))],
        out_specs=[eb(EDIM), eb(1), eb(1), eb(4), eb(1), eb(1)],
        out_shape=[jax.ShapeDtypeStruct((E, EDIM), f32),
                   jax.ShapeDtypeStruct((E, 1), f32),
                   jax.ShapeDtypeStruct((E, 1), f32),
                   jax.ShapeDtypeStruct((E, 4), f32),
                   jax.ShapeDtypeStruct((E, 1), i32),
                   jax.ShapeDtypeStruct((E, 1), i32)],
    )


@functools.cache
def _tc_layer_node():
    def body(s_ref, v_ref, sg0_ref, sg1_ref, vp0_ref, vp1_ref, icnt_ref,
             W2, b2, W1a, W1b, s_new_ref, v_new_ref, p_ref, q_ref):
        icnt = icnt_ref[...]
        seg = (sg0_ref[...] + sg1_ref[...]) * icnt
        s_new = s_ref[...] + jnp.dot(seg, W2[...], preferred_element_type=f32) + b2[...]
        s_new_ref[...] = s_new
        v_new_ref[...] = v_ref[...] + (vp0_ref[...] + vp1_ref[...]) * icnt
        p_ref[...] = jnp.dot(s_new, W1a[...], preferred_element_type=f32)
        q_ref[...] = jnp.dot(s_new, W1b[...], preferred_element_type=f32)

    return pl.pallas_call(
        body,
        out_shape=[jax.ShapeDtypeStruct((N, SDIM), f32),
                   jax.ShapeDtypeStruct((N, V3), f32),
                   jax.ShapeDtypeStruct((N, SDIM), f32),
                   jax.ShapeDtypeStruct((N, SDIM), f32)],
    )


def _tc_edge_msg_body(first, g1_ref, g2_ref, vs_ref, e_ref, d_ref, a_ref, rn_ref,
                      W1c, wd, wa, b1, We, be_, Wg, Wg2,
                      m_ref, enew_ref, vm_ref):
    pre = g1_ref[...] + g2_ref[...] \
        + jnp.dot(e_ref[...], W1c[...], preferred_element_type=f32) \
        + d_ref[...] * wd[...] + a_ref[...] * wa[...] + b1[...]
    m = _silu(pre)
    m_ref[...] = m
    enew_ref[...] = e_ref[...] + jnp.dot(m, We[...], preferred_element_type=f32) + be_[...]
    g = jnp.dot(m, Wg[...], preferred_element_type=f32)
    rn = rn_ref[...]
    vm = jnp.concatenate([rn[:, 0:1] * g, rn[:, 1:2] * g, rn[:, 2:3] * g], axis=-1)
    if not first:
        g2g = jnp.dot(m, Wg2[...], preferred_element_type=f32)
        vm = vm + vs_ref[...] * jnp.concatenate([g2g, g2g, g2g], axis=-1)
    vm_ref[...] = vm


@functools.cache
def _tc_edge_msg(first):
    grid = (E // BE,)
    eb = lambda w: pl.BlockSpec((BE, w), lambda i: (i, 0))
    wb = lambda s: pl.BlockSpec(s, lambda i: (0,) * len(s))
    in_specs = [eb(SDIM), eb(SDIM)]
    if not first:
        in_specs.append(eb(V3))
    in_specs += [eb(EDIM), eb(1), eb(1), eb(4),
                 wb((EDIM, SDIM)), wb((1, SDIM)), wb((1, SDIM)), wb((1, SDIM)),
                 wb((SDIM, EDIM)), wb((1, EDIM)),
                 wb((SDIM, VDIM)), wb((SDIM, VDIM))]

    def body(*refs):
        if first:
            g1, g2, e, d, a, rn, W1c, wd, wa, b1, We, be_, Wg, Wg2, m, en, vm = refs
            _tc_edge_msg_body(True, g1, g2, None, e, d, a, rn,
                              W1c, wd, wa, b1, We, be_, Wg, Wg2, m, en, vm)
        else:
            g1, g2, vs, e, d, a, rn, W1c, wd, wa, b1, We, be_, Wg, Wg2, m, en, vm = refs
            _tc_edge_msg_body(False, g1, g2, vs, e, d, a, rn,
                              W1c, wd, wa, b1, We, be_, Wg, Wg2, m, en, vm)

    return pl.pallas_call(
        body,
        grid=grid,
        in_specs=in_specs,
        out_specs=[eb(SDIM), eb(EDIM), eb(V3)],
        out_shape=[jax.ShapeDtypeStruct((E, SDIM), f32),
                   jax.ShapeDtypeStruct((E, EDIM), f32),
                   jax.ShapeDtypeStruct((E, V3), f32)],
    )


@functools.cache
def _tc_head_node():
    def body(s_ref, v_ref, sg0_ref, sg1_ref, vp0_ref, vp1_ref, icnt_ref,
             pos_ref, batch_ref, W2, b2, W_sh, b_sh, W_ao, b_ao, W_co,
             W_b0a, W_bm, b_bm, b_b0,
             atoms_ref, cp_ref, s2_ref, wcomb_ref, bias0_ref):
        icnt = icnt_ref[...]
        seg = (sg0_ref[...] + sg1_ref[...]) * icnt
        s5 = s_ref[...] + jnp.dot(seg, W2[...], preferred_element_type=f32) + b2[...]
        v5 = v_ref[...] + (vp0_ref[...] + vp1_ref[...]) * icnt
        sh = _silu(jnp.dot(s5, W_sh[...], preferred_element_type=f32) + b_sh[...])
        atoms_ref[...] = jnp.dot(sh, W_ao[...], preferred_element_type=f32) + b_ao[...]
        wco = W_co[...]
        c0 = jnp.sum(v5[:, 0:VDIM] * wco, axis=-1, keepdims=True)
        c1 = jnp.sum(v5[:, VDIM:2 * VDIM] * wco, axis=-1, keepdims=True)
        c2 = jnp.sum(v5[:, 2 * VDIM:] * wco, axis=-1, keepdims=True)
        cp0 = pos_ref[...] + jnp.concatenate(
            [c0, c1, c2, jnp.zeros((N, 5), f32)], axis=-1)
        oh = (batch_ref[...] == lax.broadcasted_iota(i32, (1, B), 1)).astype(f32)
        cp0e = jnp.concatenate([cp0, jnp.ones((N, 1), f32)], axis=-1)
        sums = lax.dot_general(oh, cp0e, (((0,), (0,)), ((), ())),
                               preferred_element_type=f32)
        means = sums[:, 0:8] / jnp.maximum(sums[:, 8:9], 1.0)
        cp_ref[...] = cp0 - jnp.dot(oh, means, preferred_element_type=f32)
        s2_ref[...] = jnp.dot(sh, W_b0a[...], preferred_element_type=f32)
        wcomb_ref[...] = jnp.dot(W_bm[...], W_b0a[...], preferred_element_type=f32)
        bias0_ref[...] = jnp.dot(b_bm[...], W_b0a[...],
                                 preferred_element_type=f32) + b_b0[...]

    return pl.pallas_call(
        body,
        out_shape=[jax.ShapeDtypeStruct((N, NAF), f32),
                   jax.ShapeDtypeStruct((N, 8), f32),
                   jax.ShapeDtypeStruct((N, SDIM), f32),
                   jax.ShapeDtypeStruct((EDIM, SDIM), f32),
                   jax.ShapeDtypeStruct((1, SDIM), f32)],
    )


@functools.cache
def _tc_head_edge():
    def body(s1_ref, s2_ref, ef_ref, er_ref, cpi_ref, cpj_ref,
             wcomb, bias0, wdd, W_b1, b_b1, bonds_ref):
        diff = cpi_ref[...] - cpj_ref[...]
        dd = jnp.sum(diff * diff, axis=-1, keepdims=True)
        es = 0.5 * (ef_ref[...] + er_ref[...])
        pre = s1_ref[...] + s2_ref[...] \
            + jnp.dot(es, wcomb[...], preferred_element_type=f32) \
            + dd * wdd[...] + bias0[...]
        h = _silu(pre)
        bonds_ref[...] = jnp.dot(h, W_b1[...], preferred_element_type=f32) + b_b1[...]

    grid = (E // BE,)
    eb = lambda w: pl.BlockSpec((BE, w), lambda i: (i, 0))
    wb = lambda s: pl.BlockSpec(s, lambda i: (0,) * len(s))
    return pl.pallas_call(
        body,
        grid=grid,
        in_specs=[eb(SDIM), eb(SDIM), eb(EDIM), eb(EDIM), eb(8), eb(8),
                  wb((EDIM, SDIM)), wb((1, SDIM)), wb((1, SDIM)),
                  wb((SDIM, NBT)), wb((1, NBT))],
        out_specs=[eb(NBT)],
        out_shape=[jax.ShapeDtypeStruct((E, NBT), f32)],
    )


def kernel(x, t, pos, edge_index, edge_attr, batch, params):
    src = edge_index[0]
    dst = edge_index[1]
    batch2d = batch.reshape(N, 1)
    pos4 = jnp.concatenate([pos, jnp.zeros((N, 5), f32)], axis=-1)
    src2d = src.reshape(E, 1)
    dst2d = dst.reshape(E, 1)

    W1 = params['W1']
    W1a = [W1[l, :SDIM] for l in range(NL)]
    W1b = [W1[l, SDIM:2 * SDIM] for l in range(NL)]
    W1c = [W1[l, 2 * SDIM:2 * SDIM + EDIM] for l in range(NL)]
    wd = [W1[l, 2 * SDIM + EDIM].reshape(1, SDIM) for l in range(NL)]
    wa = [W1[l, 2 * SDIM + EDIM + 1].reshape(1, SDIM) for l in range(NL)]
    b1 = [params['b1'][l].reshape(1, SDIM) for l in range(NL)]

    def tb3_body(t_ref, batch_ref, W_tb, b_tb, W_btt, b_btt, tb3_ref):
        tb2 = (t_ref[...] * W_tb[...] + b_tb[...]) @ W_btt[...] + b_btt[...]
        oh = (batch_ref[...] == lax.broadcasted_iota(i32, (1, B), 1)).astype(f32)
        tb3_ref[...] = jnp.dot(oh, tb2, preferred_element_type=f32)

    tb3 = pl.pallas_call(
        tb3_body, out_shape=jax.ShapeDtypeStruct((N, EDIM), f32),
    )(t, batch2d, params['W_tb'], params['b_tb'], params['W_btt'], params['b_btt'])

    ones_ch = jnp.ones((CH,), f32)
    zeros_n = jnp.zeros((N,), f32)
    ps4, pd4, tbg, cnt2 = _sc_pre()(pos4, tb3, src, dst, ones_ch, zeros_n)

    s0, P, Q, _tb3_unused, inv_cnt = _tc_pre_node()(
        x, t, batch2d, cnt2, params['W_atom'], params['b_atom'],
        params['W_ta'], params['b_ta'], params['W_att'], params['b_att'],
        params['W_tb'], params['b_tb'], params['W_btt'], params['b_btt'],
        W1a[0], W1b[0])

    e0, d_e, a_e, rn4, key2d, rkey2d = _tc_pre_edge()(
        edge_attr, tbg, ps4, pd4, src2d, dst2d,
        params['W_bond'], params['b_bond'].reshape(1, EDIM))

    zs = jnp.zeros((N, SDIM), f32)
    zv = jnp.zeros((N, V3), f32)

    s, e, v = s0, e0, None
    segm = segv = None
    for l in range(NL):
        if l == 0:
            G1, G2 = _sc_gather2()(P, Q, src, dst)
            m, e, vm = _tc_edge_msg(True)(
                G1, G2, e, d_e, a_e, rn4,
                W1c[0], wd[0], wa[0], b1[0],
                params['We'][0], params['be'][0].reshape(1, EDIM),
                params['Wg'][0], params['Wg2'][0])
        else:
            s, v, P, Q = _tc_layer_node()(
                s, v if v is not None else jnp.zeros((N, V3), f32),
                segm[0], segm[1], segv[0], segv[1], inv_cnt,
                params['W2'][l - 1], params['b2'][l - 1].reshape(1, SDIM),
                W1a[l], W1b[l])
            G1, G2, Vs = _sc_gather3()(P, Q, v, src, dst)
            m, e, vm = _tc_edge_msg(False)(
                G1, G2, Vs, e, d_e, a_e, rn4,
                W1c[l], wd[l], wa[l], b1[l],
                params['We'][l], params['be'][l].reshape(1, EDIM),
                params['Wg'][l], params['Wg2'][l])
        segm, segv = _sc_scat2()(m, vm, dst, zs, zv)
        if l == 0:
            v = jnp.zeros((N, V3), f32)

    atoms_pred, cp4, s2, wcomb, bias0 = _tc_head_node()(
        s, v, segm[0], segm[1], segv[0], segv[1], inv_cnt, pos4, batch2d,
        params['W2'][NL - 1], params['b2'][NL - 1].reshape(1, SDIM),
        params['W_sh'], params['b_sh'].reshape(1, SDIM),
        params['W_ao'], params['b_ao'].reshape(1, NAF),
        params['W_co'].reshape(1, VDIM),
        params['W_b0'][:SDIM], params['W_bm'],
        params['b_bm'].reshape(1, SDIM), params['b_b0'].reshape(1, SDIM))

    key = key2d.reshape(E)
    rkey = rkey2d.reshape(E)
    ids = jnp.arange(E, dtype=i32)
    neg1 = jnp.full((16384,), -1, i32)
    fwd, rev, _tbl = _sc_sym()(key, rkey, ids, neg1)

    e_ext = jnp.concatenate([e, jnp.zeros((8, EDIM), f32)], axis=0)
    S1, S2, cpi, cpj, ef, er = _sc_head()(s2, cp4, e_ext, src, dst, fwd, rev)

    bonds_pred, = _tc_head_edge()(
        S1, S2, ef, er, cpi, cpj, wcomb, bias0,
        params['W_b0'][SDIM].reshape(1, SDIM),
        params['W_b1'], params['b_b1'].reshape(1, NBT))

    coords_pred = cp4[:, :3]
    return coords_pred, atoms_pred, bonds_pred


# speedup vs baseline: 3.9275x; 2.5963x over previous
import functools

import jax
import jax.numpy as jnp
from jax import lax
from jax.experimental import pallas as pl
from jax.experimental.pallas import tpu as pltpu
from jax.experimental.pallas import tpu_sc as plsc

N = 2048; E = 32768; B = 64
NAF = 16; NBT = 5; SDIM = 256; VDIM = 64; EDIM = 32; NL = 5
V3 = 3 * VDIM

NC = 2
NS = 16
NW = NC * NS
EPW = E // NW
CH = 128
NCHUNK = EPW // CH
TBL = N * N
TPW = TBL // NW
DUMP0 = TBL

f32 = jnp.float32
i32 = jnp.int32


def _mesh():
    return plsc.VectorSubcoreMesh(core_axis_name="c", subcore_axis_name="s",
                                  num_cores=NC, num_subcores=NS)


_SC_PARAMS = pltpu.CompilerParams(use_tc_tiling_on_sc=False)


def _wid():
    return lax.axis_index("c") * NS + lax.axis_index("s")


@functools.cache
def _sc_gather2():
    @functools.partial(
        pl.kernel,
        out_type=[jax.ShapeDtypeStruct((E, SDIM), f32),
                  jax.ShapeDtypeStruct((E, SDIM), f32)],
        mesh=_mesh(),
        compiler_params=_SC_PARAMS,
        scratch_types=[pltpu.VMEM((CH,), i32), pltpu.VMEM((CH,), i32),
                       pltpu.VMEM((CH, SDIM), f32)],
    )
    def k(p_hbm, q_hbm, src_hbm, dst_hbm, g1_hbm, g2_hbm, isv, idv, rows):
        base0 = _wid() * EPW

        def body(i, _):
            off = base0 + i * CH
            pltpu.sync_copy(src_hbm.at[pl.ds(off, CH)], isv)
            pltpu.sync_copy(dst_hbm.at[pl.ds(off, CH)], idv)
            pltpu.sync_copy(p_hbm.at[isv], rows)
            pltpu.sync_copy(rows, g1_hbm.at[pl.ds(off, CH)])
            pltpu.sync_copy(q_hbm.at[idv], rows)
            pltpu.sync_copy(rows, g2_hbm.at[pl.ds(off, CH)])
            return 0

        lax.fori_loop(0, NCHUNK, body, 0)

    return k


@functools.cache
def _sc_gather3():
    @functools.partial(
        pl.kernel,
        out_type=[jax.ShapeDtypeStruct((E, SDIM), f32),
                  jax.ShapeDtypeStruct((E, SDIM), f32),
                  jax.ShapeDtypeStruct((E, V3), f32)],
        mesh=_mesh(),
        compiler_params=_SC_PARAMS,
        scratch_types=[pltpu.VMEM((CH,), i32), pltpu.VMEM((CH,), i32),
                       pltpu.VMEM((CH, SDIM), f32), pltpu.VMEM((CH, V3), f32)],
    )
    def k(p_hbm, q_hbm, v_hbm, src_hbm, dst_hbm, g1_hbm, g2_hbm, vs_hbm,
          isv, idv, rows, vrows):
        base0 = _wid() * EPW

        def body(i, _):
            off = base0 + i * CH
            pltpu.sync_copy(src_hbm.at[pl.ds(off, CH)], isv)
            pltpu.sync_copy(dst_hbm.at[pl.ds(off, CH)], idv)
            pltpu.sync_copy(p_hbm.at[isv], rows)
            pltpu.sync_copy(rows, g1_hbm.at[pl.ds(off, CH)])
            pltpu.sync_copy(q_hbm.at[idv], rows)
            pltpu.sync_copy(rows, g2_hbm.at[pl.ds(off, CH)])
            pltpu.sync_copy(v_hbm.at[isv], vrows)
            pltpu.sync_copy(vrows, vs_hbm.at[pl.ds(off, CH)])
            return 0

        lax.fori_loop(0, NCHUNK, body, 0)

    return k


@functools.cache
def _sc_pre():
    @functools.partial(
        pl.kernel,
        out_type=[jax.ShapeDtypeStruct((E, 8), f32),
                  jax.ShapeDtypeStruct((E, 8), f32),
                  jax.ShapeDtypeStruct((E, EDIM), f32),
                  jax.ShapeDtypeStruct((NC, N), f32)],
        mesh=_mesh(),
        compiler_params=_SC_PARAMS,
        scratch_types=[pltpu.VMEM((CH,), i32), pltpu.VMEM((CH,), i32),
                       pltpu.VMEM((CH, 8), f32), pltpu.VMEM((CH, EDIM), f32),
                       pltpu.VMEM((CH,), f32),
                       pltpu.VMEM_SHARED((N,), f32)],
    )
    def k(pos_hbm, tb3_hbm, src_hbm, dst_hbm, ones_hbm, zn_hbm,
          ps_hbm, pd_hbm, tbg_hbm, cnt_hbm,
          isv, idv, rows4, rows32, onesv, acc_cnt):
        cid = lax.axis_index("c")
        sid = lax.axis_index("s")
        base0 = (cid * NS + sid) * EPW

        @pl.when(sid == 0)
        def _():
            pltpu.sync_copy(zn_hbm, acc_cnt)

        pltpu.sync_copy(ones_hbm, onesv)
        plsc.subcore_barrier()

        def body(i, _):
            off = base0 + i * CH
            pltpu.sync_copy(src_hbm.at[pl.ds(off, CH)], isv)
            pltpu.sync_copy(dst_hbm.at[pl.ds(off, CH)], idv)
            pltpu.sync_copy(pos_hbm.at[isv], rows4)
            pltpu.sync_copy(rows4, ps_hbm.at[pl.ds(off, CH)])
            pltpu.sync_copy(pos_hbm.at[idv], rows4)
            pltpu.sync_copy(rows4, pd_hbm.at[pl.ds(off, CH)])
            pltpu.sync_copy(tb3_hbm.at[isv], rows32)
            pltpu.sync_copy(rows32, tbg_hbm.at[pl.ds(off, CH)])
            pltpu.sync_copy(onesv, acc_cnt.at[idv], add=True)
            return 0

        lax.fori_loop(0, NCHUNK, body, 0)
        plsc.subcore_barrier()

        @pl.when(sid == 0)
        def _():
            pltpu.sync_copy(acc_cnt, cnt_hbm.at[cid])

    return k


@functools.cache
def _sc_scat2():
    @functools.partial(
        pl.kernel,
        out_type=[jax.ShapeDtypeStruct((NC, N, SDIM), f32),
                  jax.ShapeDtypeStruct((NC, N, V3), f32)],
        mesh=_mesh(),
        compiler_params=_SC_PARAMS,
        scratch_types=[pltpu.VMEM((CH,), i32),
                       pltpu.VMEM((CH, SDIM), f32), pltpu.VMEM((CH, V3), f32),
                       pltpu.VMEM_SHARED((N, SDIM), f32),
                       pltpu.VMEM_SHARED((N, V3), f32)],
    )
    def k(m_hbm, vm_hbm, dst_hbm, zs_hbm, zv_hbm, segm_hbm, segv_hbm,
          idv, mrows, vrows, acc_s, acc_v):
        cid = lax.axis_index("c")
        sid = lax.axis_index("s")
        base0 = (cid * NS + sid) * EPW
        npt = N // NS

        pltpu.sync_copy(zs_hbm.at[pl.ds(sid * npt, npt)], acc_s.at[pl.ds(sid * npt, npt)])
        pltpu.sync_copy(zv_hbm.at[pl.ds(sid * npt, npt)], acc_v.at[pl.ds(sid * npt, npt)])
        plsc.subcore_barrier()

        def body(i, _):
            off = base0 + i * CH
            pltpu.sync_copy(dst_hbm.at[pl.ds(off, CH)], idv)
            pltpu.sync_copy(m_hbm.at[pl.ds(off, CH)], mrows)
            pltpu.sync_copy(vm_hbm.at[pl.ds(off, CH)], vrows)
            pltpu.sync_copy(mrows, acc_s.at[idv], add=True)
            pltpu.sync_copy(vrows, acc_v.at[idv], add=True)
            return 0

        lax.fori_loop(0, NCHUNK, body, 0)
        plsc.subcore_barrier()

        pltpu.sync_copy(acc_s.at[pl.ds(sid * npt, npt)], segm_hbm.at[cid, pl.ds(sid * npt, npt)])
        pltpu.sync_copy(acc_v.at[pl.ds(sid * npt, npt)], segv_hbm.at[cid, pl.ds(sid * npt, npt)])

    return k


@functools.cache
def _sc_sym():
    @functools.partial(
        pl.kernel,
        out_type=[jax.ShapeDtypeStruct((E,), i32),
                  jax.ShapeDtypeStruct((E,), i32),
                  jax.ShapeDtypeStruct((TBL + 512,), i32)],
        mesh=_mesh(),
        compiler_params=_SC_PARAMS,
        scratch_types=[pltpu.VMEM((CH,), i32), pltpu.VMEM((CH,), i32),
                       pltpu.VMEM((CH,), i32), pltpu.VMEM((CH,), i32),
                       pltpu.VMEM((16384,), i32)],
    )
    def k(key_hbm, rkey_hbm, ids_hbm, neg_hbm, fwd_hbm, rev_hbm, tbl_hbm,
          keyv, idsv, tv, idx2, negv):
        w = _wid()
        base0 = w * EPW

        tb = w * TPW
        pltpu.sync_copy(neg_hbm, negv)
        def mset(i, _):
            pltpu.sync_copy(negv, tbl_hbm.at[pl.ds(tb + i * 16384, 16384)])
            return 0
        lax.fori_loop(0, TPW // 16384, mset, 0)

        plsc.subcore_barrier()

        def scat(i, _):
            off = base0 + i * CH
            pltpu.sync_copy(key_hbm.at[pl.ds(off, CH)], keyv)
            pltpu.sync_copy(ids_hbm.at[pl.ds(off, CH)], idsv)
            pltpu.sync_copy(idsv, tbl_hbm.at[keyv])
            return 0
        lax.fori_loop(0, NCHUNK, scat, 0)
        plsc.subcore_barrier()

        dump = DUMP0 + w * 16
        def fix(_p, __):
            def body(i, _):
                off = base0 + i * CH
                pltpu.sync_copy(key_hbm.at[pl.ds(off, CH)], keyv)
                pltpu.sync_copy(ids_hbm.at[pl.ds(off, CH)], idsv)
                pltpu.sync_copy(tbl_hbm.at[keyv], tv)
                def sel(j, _):
                    kv = keyv[pl.ds(j * 16, 16)]
                    iv = idsv[pl.ds(j * 16, 16)]
                    t = tv[pl.ds(j * 16, 16)]
                    idx2[pl.ds(j * 16, 16)] = jnp.where(iv > t, kv, dump)
                    return 0
                lax.fori_loop(0, CH // 16, sel, 0)
                pltpu.sync_copy(idsv, tbl_hbm.at[idx2])
                return 0
            lax.fori_loop(0, NCHUNK, body, 0)
            plsc.subcore_barrier()
            return 0
        lax.fori_loop(0, 3, fix, 0)

        def fin(i, _):
            off = base0 + i * CH
            pltpu.sync_copy(key_hbm.at[pl.ds(off, CH)], keyv)
            pltpu.sync_copy(tbl_hbm.at[keyv], tv)
            pltpu.sync_copy(tv, fwd_hbm.at[pl.ds(off, CH)])
            pltpu.sync_copy(rkey_hbm.at[pl.ds(off, CH)], keyv)
            pltpu.sync_copy(tbl_hbm.at[keyv], tv)
            pltpu.sync_copy(tv, rev_hbm.at[pl.ds(off, CH)])
            return 0
        lax.fori_loop(0, NCHUNK, fin, 0)

    return k


@functools.cache
def _sc_head():
    @functools.partial(
        pl.kernel,
        out_type=[jax.ShapeDtypeStruct((E, SDIM), f32),
                  jax.ShapeDtypeStruct((E, SDIM), f32),
                  jax.ShapeDtypeStruct((E, 8), f32),
                  jax.ShapeDtypeStruct((E, 8), f32),
                  jax.ShapeDtypeStruct((E, EDIM), f32),
                  jax.ShapeDtypeStruct((E, EDIM), f32)],
        mesh=_mesh(),
        compiler_params=_SC_PARAMS,
        scratch_types=[pltpu.VMEM((CH,), i32), pltpu.VMEM((CH,), i32),
                       pltpu.VMEM((CH,), i32),
                       pltpu.VMEM((CH, SDIM), f32), pltpu.VMEM((CH, 8), f32),
                       pltpu.VMEM((CH, EDIM), f32)],
    )
    def k(s2_hbm, cp_hbm, eext_hbm, src_hbm, dst_hbm, fwd_hbm, rev_hbm,
          s1o, s2o, cpio, cpjo, efo, ero,
          isv, idv, iwv, rows, rows4, rows32):
        base0 = _wid() * EPW

        def body(i, _):
            off = base0 + i * CH
            pltpu.sync_copy(src_hbm.at[pl.ds(off, CH)], isv)
            pltpu.sync_copy(dst_hbm.at[pl.ds(off, CH)], idv)
            pltpu.sync_copy(s2_hbm.at[idv], rows)
            pltpu.sync_copy(rows, s1o.at[pl.ds(off, CH)])
            pltpu.sync_copy(s2_hbm.at[isv], rows)
            pltpu.sync_copy(rows, s2o.at[pl.ds(off, CH)])
            pltpu.sync_copy(cp_hbm.at[idv], rows4)
            pltpu.sync_copy(rows4, cpio.at[pl.ds(off, CH)])
            pltpu.sync_copy(cp_hbm.at[isv], rows4)
            pltpu.sync_copy(rows4, cpjo.at[pl.ds(off, CH)])
            pltpu.sync_copy(fwd_hbm.at[pl.ds(off, CH)], iwv)
            pltpu.sync_copy(eext_hbm.at[iwv], rows32)
            pltpu.sync_copy(rows32, efo.at[pl.ds(off, CH)])
            pltpu.sync_copy(rev_hbm.at[pl.ds(off, CH)], iwv)
            def sel(j, _):
                rv = iwv[pl.ds(j * 16, 16)]
                iwv[pl.ds(j * 16, 16)] = jnp.where(rv < 0, E, rv)
                return 0
            lax.fori_loop(0, CH // 16, sel, 0)
            pltpu.sync_copy(eext_hbm.at[iwv], rows32)
            pltpu.sync_copy(rows32, ero.at[pl.ds(off, CH)])
            return 0

        lax.fori_loop(0, NCHUNK, body, 0)

    return k


def _silu(x):
    return x * jax.nn.sigmoid(x)


@functools.cache
def _tc_pre_node():
    def body(x_ref, t_ref, batch_ref, cnt_ref, W_atom, b_atom, W_ta, b_ta,
             W_att, b_att, W_tb, b_tb, W_btt, b_btt, W1a, W1b,
             s_ref, p_ref, q_ref, tb3_ref, icnt_ref):
        t = t_ref[...]
        ta2 = jnp.dot(t * W_ta[...] + b_ta[...], W_att[...],
                      preferred_element_type=f32) + b_att[...]
        tb2 = (t * W_tb[...] + b_tb[...]) @ W_btt[...] + b_btt[...]
        oh = (batch_ref[...] == lax.broadcasted_iota(i32, (1, B), 1)).astype(f32)
        s0 = jnp.dot(x_ref[...], W_atom[...], preferred_element_type=f32) \
            + b_atom[...] + jnp.dot(oh, ta2, preferred_element_type=f32)
        s_ref[...] = s0
        tb3_ref[...] = jnp.dot(oh, tb2, preferred_element_type=f32)
        p_ref[...] = jnp.dot(s0, W1a[...], preferred_element_type=f32)
        q_ref[...] = jnp.dot(s0, W1b[...], preferred_element_type=f32)
        cnt = cnt_ref[0, :] + cnt_ref[1, :]
        icnt_ref[...] = (1.0 / jnp.maximum(cnt, 1.0)).reshape(N, 1)

    return pl.pallas_call(
        body,
        out_shape=[jax.ShapeDtypeStruct((N, SDIM), f32),
                   jax.ShapeDtypeStruct((N, SDIM), f32),
                   jax.ShapeDtypeStruct((N, SDIM), f32),
                   jax.ShapeDtypeStruct((N, EDIM), f32),
                   jax.ShapeDtypeStruct((N, 1), f32)],
    )


BE = 1024


@functools.cache
def _tc_pre_edge():
    def body(ea_ref, tbg_ref, ps_ref, pd_ref, src_ref, dst_ref, W_bond, b_bond,
             e_ref, d_ref, a_ref, rn_ref, key_ref, rkey_ref):
        ps = ps_ref[...]
        pd = pd_ref[...]
        e_ref[...] = jnp.dot(ea_ref[...], W_bond[...], preferred_element_type=f32) \
            + b_bond[...] + tbg_ref[...]
        r = pd - ps
        rr = jnp.sum(r * r, axis=-1, keepdims=True)
        d = jnp.sqrt(jnp.clip(rr, 1e-6, None))
        d_ref[...] = d
        ns = jnp.sqrt(jnp.sum(ps * ps, axis=-1, keepdims=True))
        nd = jnp.sqrt(jnp.sum(pd * pd, axis=-1, keepdims=True))
        pns = jnp.where(ns != 0.0, ps / jnp.where(ns == 0.0, 1.0, ns), 0.0)
        pnd = jnp.where(nd != 0.0, pd / jnp.where(nd == 0.0, 1.0, nd), 0.0)
        a_ref[...] = jnp.sum(pns * pnd, axis=-1, keepdims=True)
        rn_ref[...] = (r / (1.0 + d))[:, 0:4]
        key_ref[...] = src_ref[...] * N + dst_ref[...]
        rkey_ref[...] = dst_ref[...] * N + src_ref[...]

    grid = (E // BE,)
    eb = lambda w: pl.BlockSpec((BE, w), lambda i: (i, 0))
    wb = lambda s: pl.BlockSpec(s, lambda i: (0,) * len(s))
    return pl.pallas_call(
        body,
        grid=grid,
        in_specs=[eb(NBT), eb(EDIM), eb(8), eb(8), eb(1), eb(1),
                  wb((NBT, EDIM)), wb((1, EDIM))],
        out_specs=[eb(EDIM), eb(1), eb(1), eb(4), eb(1), eb(1)],
        out_shape=[jax.ShapeDtypeStruct((E, EDIM), f32),
                   jax.ShapeDtypeStruct((E, 1), f32),
                   jax.ShapeDtypeStruct((E, 1), f32),
                   jax.ShapeDtypeStruct((E, 4), f32),
                   jax.ShapeDtypeStruct((E, 1), i32),
                   jax.ShapeDtypeStruct((E, 1), i32)],
    )


@functools.cache
def _tc_layer_node():
    def body(s_ref, v_ref, sg0_ref, sg1_ref, vp0_ref, vp1_ref, icnt_ref,
             W2, b2, W1a, W1b, s_new_ref, v_new_ref, p_ref, q_ref):
        icnt = icnt_ref[...]
        seg = (sg0_ref[...] + sg1_ref[...]) * icnt
        s_new = s_ref[...] + jnp.dot(seg, W2[...], preferred_element_type=f32) + b2[...]
        s_new_ref[...] = s_new
        v_new_ref[...] = v_ref[...] + (vp0_ref[...] + vp1_ref[...]) * icnt
        p_ref[...] = jnp.dot(s_new, W1a[...], preferred_element_type=f32)
        q_ref[...] = jnp.dot(s_new, W1b[...], preferred_element_type=f32)

    return pl.pallas_call(
        body,
        out_shape=[jax.ShapeDtypeStruct((N, SDIM), f32),
                   jax.ShapeDtypeStruct((N, V3), f32),
                   jax.ShapeDtypeStruct((N, SDIM), f32),
                   jax.ShapeDtypeStruct((N, SDIM), f32)],
    )


def _tc_edge_msg_body(first, g1_ref, g2_ref, vs_ref, e_ref, d_ref, a_ref, rn_ref,
                      W1c, wd, wa, b1, We, be_, Wg, Wg2,
                      m_ref, enew_ref, vm_ref):
    pre = g1_ref[...] + g2_ref[...] \
        + jnp.dot(e_ref[...], W1c[...], preferred_element_type=f32) \
        + d_ref[...] * wd[...] + a_ref[...] * wa[...] + b1[...]
    m = _silu(pre)
    m_ref[...] = m
    enew_ref[...] = e_ref[...] + jnp.dot(m, We[...], preferred_element_type=f32) + be_[...]
    g = jnp.dot(m, Wg[...], preferred_element_type=f32)
    rn = rn_ref[...]
    vm = jnp.concatenate([rn[:, 0:1] * g, rn[:, 1:2] * g, rn[:, 2:3] * g], axis=-1)
    if not first:
        g2g = jnp.dot(m, Wg2[...], preferred_element_type=f32)
        vm = vm + vs_ref[...] * jnp.concatenate([g2g, g2g, g2g], axis=-1)
    vm_ref[...] = vm


@functools.cache
def _tc_edge_msg(first):
    grid = (E // BE,)
    eb = lambda w: pl.BlockSpec((BE, w), lambda i: (i, 0))
    wb = lambda s: pl.BlockSpec(s, lambda i: (0,) * len(s))
    in_specs = [eb(SDIM), eb(SDIM)]
    if not first:
        in_specs.append(eb(V3))
    in_specs += [eb(EDIM), eb(1), eb(1), eb(4),
                 wb((EDIM, SDIM)), wb((1, SDIM)), wb((1, SDIM)), wb((1, SDIM)),
                 wb((SDIM, EDIM)), wb((1, EDIM)),
                 wb((SDIM, VDIM)), wb((SDIM, VDIM))]

    def body(*refs):
        if first:
            g1, g2, e, d, a, rn, W1c, wd, wa, b1, We, be_, Wg, Wg2, m, en, vm = refs
            _tc_edge_msg_body(True, g1, g2, None, e, d, a, rn,
                              W1c, wd, wa, b1, We, be_, Wg, Wg2, m, en, vm)
        else:
            g1, g2, vs, e, d, a, rn, W1c, wd, wa, b1, We, be_, Wg, Wg2, m, en, vm = refs
            _tc_edge_msg_body(False, g1, g2, vs, e, d, a, rn,
                              W1c, wd, wa, b1, We, be_, Wg, Wg2, m, en, vm)

    return pl.pallas_call(
        body,
        grid=grid,
        in_specs=in_specs,
        out_specs=[eb(SDIM), eb(EDIM), eb(V3)],
        out_shape=[jax.ShapeDtypeStruct((E, SDIM), f32),
                   jax.ShapeDtypeStruct((E, EDIM), f32),
                   jax.ShapeDtypeStruct((E, V3), f32)],
    )


@functools.cache
def _tc_head_node():
    def body(s_ref, v_ref, sg0_ref, sg1_ref, vp0_ref, vp1_ref, icnt_ref,
             pos_ref, batch_ref, W2, b2, W_sh, b_sh, W_ao, b_ao, W_co,
             W_b0a, W_bm, b_bm, b_b0,
             atoms_ref, cp_ref, s2_ref, wcomb_ref, bias0_ref):
        icnt = icnt_ref[...]
        seg = (sg0_ref[...] + sg1_ref[...]) * icnt
        s5 = s_ref[...] + jnp.dot(seg, W2[...], preferred_element_type=f32) + b2[...]
        v5 = v_ref[...] + (vp0_ref[...] + vp1_ref[...]) * icnt
        sh = _silu(jnp.dot(s5, W_sh[...], preferred_element_type=f32) + b_sh[...])
        atoms_ref[...] = jnp.dot(sh, W_ao[...], preferred_element_type=f32) + b_ao[...]
        wco = W_co[...]
        c0 = jnp.sum(v5[:, 0:VDIM] * wco, axis=-1, keepdims=True)
        c1 = jnp.sum(v5[:, VDIM:2 * VDIM] * wco, axis=-1, keepdims=True)
        c2 = jnp.sum(v5[:, 2 * VDIM:] * wco, axis=-1, keepdims=True)
        cp0 = pos_ref[...] + jnp.concatenate(
            [c0, c1, c2, jnp.zeros((N, 5), f32)], axis=-1)
        oh = (batch_ref[...] == lax.broadcasted_iota(i32, (1, B), 1)).astype(f32)
        cp0e = jnp.concatenate([cp0, jnp.ones((N, 1), f32)], axis=-1)
        sums = lax.dot_general(oh, cp0e, (((0,), (0,)), ((), ())),
                               preferred_element_type=f32)
        means = sums[:, 0:8] / jnp.maximum(sums[:, 8:9], 1.0)
        cp_ref[...] = cp0 - jnp.dot(oh, means, preferred_element_type=f32)
        s2_ref[...] = jnp.dot(sh, W_b0a[...], preferred_element_type=f32)
        wcomb_ref[...] = jnp.dot(W_bm[...], W_b0a[...], preferred_element_type=f32)
        bias0_ref[...] = jnp.dot(b_bm[...], W_b0a[...],
                                 preferred_element_type=f32) + b_b0[...]

    return pl.pallas_call(
        body,
        out_shape=[jax.ShapeDtypeStruct((N, NAF), f32),
                   jax.ShapeDtypeStruct((N, 8), f32),
                   jax.ShapeDtypeStruct((N, SDIM), f32),
                   jax.ShapeDtypeStruct((EDIM, SDIM), f32),
                   jax.ShapeDtypeStruct((1, SDIM), f32)],
    )


@functools.cache
def _tc_head_edge():
    def body(s1_ref, s2_ref, ef_ref, er_ref, cpi_ref, cpj_ref,
             wcomb, bias0, wdd, W_b1, b_b1, bonds_ref):
        diff = cpi_ref[...] - cpj_ref[...]
        dd = jnp.sum(diff * diff, axis=-1, keepdims=True)
        es = 0.5 * (ef_ref[...] + er_ref[...])
        pre = s1_ref[...] + s2_ref[...] \
            + jnp.dot(es, wcomb[...], preferred_element_type=f32) \
            + dd * wdd[...] + bias0[...]
        h = _silu(pre)
        bonds_ref[...] = jnp.dot(h, W_b1[...], preferred_element_type=f32) + b_b1[...]

    grid = (E // BE,)
    eb = lambda w: pl.BlockSpec((BE, w), lambda i: (i, 0))
    wb = lambda s: pl.BlockSpec(s, lambda i: (0,) * len(s))
    return pl.pallas_call(
        body,
        grid=grid,
        in_specs=[eb(SDIM), eb(SDIM), eb(EDIM), eb(EDIM), eb(8), eb(8),
                  wb((EDIM, SDIM)), wb((1, SDIM)), wb((1, SDIM)),
                  wb((SDIM, NBT)), wb((1, NBT))],
        out_specs=[eb(NBT)],
        out_shape=[jax.ShapeDtypeStruct((E, NBT), f32)],
    )


def kernel(x, t, pos, edge_index, edge_attr, batch, params):
    src = edge_index[0]
    dst = edge_index[1]
    batch2d = batch.reshape(N, 1)
    pos4 = jnp.concatenate([pos, jnp.zeros((N, 5), f32)], axis=-1)
    src2d = src.reshape(E, 1)
    dst2d = dst.reshape(E, 1)

    W1 = params['W1']
    W1a = [W1[l, :SDIM] for l in range(NL)]
    W1b = [W1[l, SDIM:2 * SDIM] for l in range(NL)]
    W1c = [W1[l, 2 * SDIM:2 * SDIM + EDIM] for l in range(NL)]
    wd = [W1[l, 2 * SDIM + EDIM].reshape(1, SDIM) for l in range(NL)]
    wa = [W1[l, 2 * SDIM + EDIM + 1].reshape(1, SDIM) for l in range(NL)]
    b1 = [params['b1'][l].reshape(1, SDIM) for l in range(NL)]

    def tb3_body(t_ref, batch_ref, W_tb, b_tb, W_btt, b_btt, tb3_ref):
        tb2 = (t_ref[...] * W_tb[...] + b_tb[...]) @ W_btt[...] + b_btt[...]
        oh = (batch_ref[...] == lax.broadcasted_iota(i32, (1, B), 1)).astype(f32)
        tb3_ref[...] = jnp.dot(oh, tb2, preferred_element_type=f32)

    tb3 = pl.pallas_call(
        tb3_body, out_shape=jax.ShapeDtypeStruct((N, EDIM), f32),
    )(t, batch2d, params['W_tb'], params['b_tb'], params['W_btt'], params['b_btt'])

    ones_ch = jnp.ones((CH,), f32)
    zeros_n = jnp.zeros((N,), f32)
    ps4, pd4, tbg, cnt2 = _sc_pre()(pos4, tb3, src, dst, ones_ch, zeros_n)

    s0, P, Q, _tb3_unused, inv_cnt = _tc_pre_node()(
        x, t, batch2d, cnt2, params['W_atom'], params['b_atom'],
        params['W_ta'], params['b_ta'], params['W_att'], params['b_att'],
        params['W_tb'], params['b_tb'], params['W_btt'], params['b_btt'],
        W1a[0], W1b[0])

    e0, d_e, a_e, rn4, key2d, rkey2d = _tc_pre_edge()(
        edge_attr, tbg, ps4, pd4, src2d, dst2d,
        params['W_bond'], params['b_bond'].reshape(1, EDIM))

    zs = jnp.zeros((N, SDIM), f32)
    zv = jnp.zeros((N, V3), f32)

    s, e, v = s0, e0, None
    segm = segv = None
    for l in range(NL):
        if l == 0:
            G1, G2 = _sc_gather2()(P, Q, src, dst)
            m, e, vm = _tc_edge_msg(True)(
                G1, G2, e, d_e, a_e, rn4,
                W1c[0], wd[0], wa[0], b1[0],
                params['We'][0], params['be'][0].reshape(1, EDIM),
                params['Wg'][0], params['Wg2'][0])
        else:
            s, v, P, Q = _tc_layer_node()(
                s, v if v is not None else jnp.zeros((N, V3), f32),
                segm[0], segm[1], segv[0], segv[1], inv_cnt,
                params['W2'][l - 1], params['b2'][l - 1].reshape(1, SDIM),
                W1a[l], W1b[l])
            G1, G2, Vs = _sc_gather3()(P, Q, v, src, dst)
            m, e, vm = _tc_edge_msg(False)(
                G1, G2, Vs, e, d_e, a_e, rn4,
                W1c[l], wd[l], wa[l], b1[l],
                params['We'][l], params['be'][l].reshape(1, EDIM),
                params['Wg'][l], params['Wg2'][l])
        segm, segv = _sc_scat2()(m, vm, dst, zs, zv)
        if l == 0:
            v = jnp.zeros((N, V3), f32)

    atoms_pred, cp4, s2, wcomb, bias0 = _tc_head_node()(
        s, v, segm[0], segm[1], segv[0], segv[1], inv_cnt, pos4, batch2d,
        params['W2'][NL - 1], params['b2'][NL - 1].reshape(1, SDIM),
        params['W_sh'], params['b_sh'].reshape(1, SDIM),
        params['W_ao'], params['b_ao'].reshape(1, NAF),
        params['W_co'].reshape(1, VDIM),
        params['W_b0'][:SDIM], params['W_bm'],
        params['b_bm'].reshape(1, SDIM), params['b_b0'].reshape(1, SDIM))

    key = key2d.reshape(E)
    rkey = rkey2d.reshape(E)
    ids = jnp.arange(E, dtype=i32)
    neg1 = jnp.full((16384,), -1, i32)
    fwd, rev, _tbl = _sc_sym()(key, rkey, ids, neg1)

    e_ext = jnp.concatenate([e, jnp.zeros((8, EDIM), f32)], axis=0)
    S1, S2, cpi, cpj, ef, er = _sc_head()(s2, cp4, e_ext, src, dst, fwd, rev)

    bonds_pred, = _tc_head_edge()(
        S1, S2, ef, er, cpi, cpj, wcomb, bias0,
        params['W_b0'][SDIM].reshape(1, SDIM),
        params['W_b1'], params['b_b1'].reshape(1, NBT))

    coords_pred = cp4[:, :3]
    return coords_pred, atoms_pred, bonds_pred


# speedup vs baseline: 7.0322x; 1.7905x over previous
import functools

import jax
import jax.numpy as jnp
from jax import lax
from jax.experimental import pallas as pl
from jax.experimental.pallas import tpu as pltpu
from jax.experimental.pallas import tpu_sc as plsc

N = 2048; E = 32768; B = 64
NAF = 16; NBT = 5; SDIM = 256; VDIM = 64; EDIM = 32; NL = 5
V3 = 3 * VDIM

NC = 2
NS = 16
NW = NC * NS
EPW = E // NW
CH = 128
NCHUNK = EPW // CH
TBL = N * N
TPW = TBL // NW
DUMP0 = TBL

f32 = jnp.float32
i32 = jnp.int32


def _mesh():
    return plsc.VectorSubcoreMesh(core_axis_name="c", subcore_axis_name="s",
                                  num_cores=NC, num_subcores=NS)


_SC_PARAMS = pltpu.CompilerParams(use_tc_tiling_on_sc=False)


def _wid():
    return lax.axis_index("c") * NS + lax.axis_index("s")


@functools.cache
def _sc_gather2():
    @functools.partial(
        pl.kernel,
        out_type=[jax.ShapeDtypeStruct((E, SDIM), f32),
                  jax.ShapeDtypeStruct((E, SDIM), f32)],
        mesh=_mesh(),
        compiler_params=_SC_PARAMS,
        scratch_types=[pltpu.VMEM((CH,), i32), pltpu.VMEM((CH,), i32),
                       pltpu.VMEM((CH, SDIM), f32)],
    )
    def k(p_hbm, q_hbm, src_hbm, dst_hbm, g1_hbm, g2_hbm, isv, idv, rows):
        base0 = _wid() * EPW

        def body(i, _):
            off = base0 + i * CH
            pltpu.sync_copy(src_hbm.at[pl.ds(off, CH)], isv)
            pltpu.sync_copy(dst_hbm.at[pl.ds(off, CH)], idv)
            pltpu.sync_copy(p_hbm.at[isv], rows)
            pltpu.sync_copy(rows, g1_hbm.at[pl.ds(off, CH)])
            pltpu.sync_copy(q_hbm.at[idv], rows)
            pltpu.sync_copy(rows, g2_hbm.at[pl.ds(off, CH)])
            return 0

        lax.fori_loop(0, NCHUNK, body, 0)

    return k


@functools.cache
def _sc_gather3():
    @functools.partial(
        pl.kernel,
        out_type=[jax.ShapeDtypeStruct((E, SDIM), f32),
                  jax.ShapeDtypeStruct((E, SDIM), f32),
                  jax.ShapeDtypeStruct((E, V3), f32)],
        mesh=_mesh(),
        compiler_params=_SC_PARAMS,
        scratch_types=[pltpu.VMEM((CH,), i32), pltpu.VMEM((CH,), i32),
                       pltpu.VMEM((CH, SDIM), f32), pltpu.VMEM((CH, V3), f32)],
    )
    def k(p_hbm, q_hbm, v_hbm, src_hbm, dst_hbm, g1_hbm, g2_hbm, vs_hbm,
          isv, idv, rows, vrows):
        base0 = _wid() * EPW

        def body(i, _):
            off = base0 + i * CH
            pltpu.sync_copy(src_hbm.at[pl.ds(off, CH)], isv)
            pltpu.sync_copy(dst_hbm.at[pl.ds(off, CH)], idv)
            pltpu.sync_copy(p_hbm.at[isv], rows)
            pltpu.sync_copy(rows, g1_hbm.at[pl.ds(off, CH)])
            pltpu.sync_copy(q_hbm.at[idv], rows)
            pltpu.sync_copy(rows, g2_hbm.at[pl.ds(off, CH)])
            pltpu.sync_copy(v_hbm.at[isv], vrows)
            pltpu.sync_copy(vrows, vs_hbm.at[pl.ds(off, CH)])
            return 0

        lax.fori_loop(0, NCHUNK, body, 0)

    return k


@functools.cache
def _sc_pre():
    @functools.partial(
        pl.kernel,
        out_type=[jax.ShapeDtypeStruct((E, 8), f32),
                  jax.ShapeDtypeStruct((E, 8), f32),
                  jax.ShapeDtypeStruct((E, EDIM), f32),
                  jax.ShapeDtypeStruct((NC, N), f32)],
        mesh=_mesh(),
        compiler_params=_SC_PARAMS,
        scratch_types=[pltpu.VMEM((CH,), i32), pltpu.VMEM((CH,), i32),
                       pltpu.VMEM((CH, 8), f32), pltpu.VMEM((CH, EDIM), f32),
                       pltpu.VMEM((CH,), f32),
                       pltpu.VMEM_SHARED((N,), f32)],
    )
    def k(pos_hbm, tb3_hbm, src_hbm, dst_hbm, ones_hbm, zn_hbm,
          ps_hbm, pd_hbm, tbg_hbm, cnt_hbm,
          isv, idv, rows4, rows32, onesv, acc_cnt):
        cid = lax.axis_index("c")
        sid = lax.axis_index("s")
        base0 = (cid * NS + sid) * EPW

        @pl.when(sid == 0)
        def _():
            pltpu.sync_copy(zn_hbm, acc_cnt)

        pltpu.sync_copy(ones_hbm, onesv)
        plsc.subcore_barrier()

        def body(i, _):
            off = base0 + i * CH
            pltpu.sync_copy(src_hbm.at[pl.ds(off, CH)], isv)
            pltpu.sync_copy(dst_hbm.at[pl.ds(off, CH)], idv)
            pltpu.sync_copy(pos_hbm.at[isv], rows4)
            pltpu.sync_copy(rows4, ps_hbm.at[pl.ds(off, CH)])
            pltpu.sync_copy(pos_hbm.at[idv], rows4)
            pltpu.sync_copy(rows4, pd_hbm.at[pl.ds(off, CH)])
            pltpu.sync_copy(tb3_hbm.at[isv], rows32)
            pltpu.sync_copy(rows32, tbg_hbm.at[pl.ds(off, CH)])
            pltpu.sync_copy(onesv, acc_cnt.at[idv], add=True)
            return 0

        lax.fori_loop(0, NCHUNK, body, 0)
        plsc.subcore_barrier()

        @pl.when(sid == 0)
        def _():
            pltpu.sync_copy(acc_cnt, cnt_hbm.at[cid])

    return k


@functools.cache
def _sc_scat2():
    @functools.partial(
        pl.kernel,
        out_type=[jax.ShapeDtypeStruct((NC, N, SDIM), f32),
                  jax.ShapeDtypeStruct((NC, N, V3), f32)],
        mesh=_mesh(),
        compiler_params=_SC_PARAMS,
        scratch_types=[pltpu.VMEM((CH,), i32),
                       pltpu.VMEM((CH, SDIM), f32), pltpu.VMEM((CH, V3), f32),
                       pltpu.VMEM_SHARED((N, SDIM), f32),
                       pltpu.VMEM_SHARED((N, V3), f32)],
    )
    def k(m_hbm, vm_hbm, dst_hbm, zs_hbm, zv_hbm, segm_hbm, segv_hbm,
          idv, mrows, vrows, acc_s, acc_v):
        cid = lax.axis_index("c")
        sid = lax.axis_index("s")
        base0 = (cid * NS + sid) * EPW
        npt = N // NS

        pltpu.sync_copy(zs_hbm.at[pl.ds(sid * npt, npt)], acc_s.at[pl.ds(sid * npt, npt)])
        pltpu.sync_copy(zv_hbm.at[pl.ds(sid * npt, npt)], acc_v.at[pl.ds(sid * npt, npt)])
        plsc.subcore_barrier()

        def body(i, _):
            off = base0 + i * CH
            pltpu.sync_copy(dst_hbm.at[pl.ds(off, CH)], idv)
            pltpu.sync_copy(m_hbm.at[pl.ds(off, CH)], mrows)
            pltpu.sync_copy(vm_hbm.at[pl.ds(off, CH)], vrows)
            pltpu.sync_copy(mrows, acc_s.at[idv], add=True)
            pltpu.sync_copy(vrows, acc_v.at[idv], add=True)
            return 0

        lax.fori_loop(0, NCHUNK, body, 0)
        plsc.subcore_barrier()

        pltpu.sync_copy(acc_s.at[pl.ds(sid * npt, npt)], segm_hbm.at[cid, pl.ds(sid * npt, npt)])
        pltpu.sync_copy(acc_v.at[pl.ds(sid * npt, npt)], segv_hbm.at[cid, pl.ds(sid * npt, npt)])

    return k


@functools.cache
def _sc_sym():
    @functools.partial(
        pl.kernel,
        out_type=[jax.ShapeDtypeStruct((E,), i32),
                  jax.ShapeDtypeStruct((E,), i32),
                  jax.ShapeDtypeStruct((TBL + 8 * CH * NW,), i32)],
        mesh=_mesh(),
        compiler_params=_SC_PARAMS,
        scratch_types=[pltpu.VMEM((CH,), i32), pltpu.VMEM((CH,), i32),
                       pltpu.VMEM((CH,), i32), pltpu.VMEM((CH,), i32),
                       pltpu.VMEM((16384,), i32)],
    )
    def k(key_hbm, rkey_hbm, ids_hbm, neg_hbm, fwd_hbm, rev_hbm, tbl_hbm,
          keyv, idsv, tv, idx2, negv):
        w = _wid()
        base0 = w * EPW

        tb = w * TPW
        pltpu.sync_copy(neg_hbm, negv)
        def mset(i, _):
            pltpu.sync_copy(negv, tbl_hbm.at[pl.ds(tb + i * 16384, 16384)])
            return 0
        lax.fori_loop(0, TPW // 16384, mset, 0)

        plsc.subcore_barrier()

        def scat(i, _):
            off = base0 + i * CH
            pltpu.sync_copy(key_hbm.at[pl.ds(off, CH)], keyv)
            pltpu.sync_copy(ids_hbm.at[pl.ds(off, CH)], idsv)
            pltpu.sync_copy(idsv, tbl_hbm.at[keyv])
            return 0
        lax.fori_loop(0, NCHUNK, scat, 0)
        plsc.subcore_barrier()

        dump = DUMP0 + w * CH
        def fix(_p, __):
            def body(i, _):
                off = base0 + i * CH
                pltpu.sync_copy(key_hbm.at[pl.ds(off, CH)], keyv)
                pltpu.sync_copy(ids_hbm.at[pl.ds(off, CH)], idsv)
                pltpu.sync_copy(tbl_hbm.at[keyv], tv)
                def sel(j, _):
                    kv = keyv[pl.ds(j * 16, 16)]
                    iv = idsv[pl.ds(j * 16, 16)]
                    t = tv[pl.ds(j * 16, 16)]
                    lane = lax.iota(i32, 16)
                    idx2[pl.ds(j * 16, 16)] = jnp.where(iv > t, kv, dump + j * 16 + lane)
                    return 0
                lax.fori_loop(0, CH // 16, sel, 0)
                pltpu.sync_copy(idsv, tbl_hbm.at[idx2])
                return 0
            lax.fori_loop(0, NCHUNK, body, 0)
            plsc.subcore_barrier()
            return 0
        lax.fori_loop(0, 3, fix, 0)

        def fin(i, _):
            off = base0 + i * CH
            pltpu.sync_copy(key_hbm.at[pl.ds(off, CH)], keyv)
            pltpu.sync_copy(tbl_hbm.at[keyv], tv)
            pltpu.sync_copy(tv, fwd_hbm.at[pl.ds(off, CH)])
            pltpu.sync_copy(rkey_hbm.at[pl.ds(off, CH)], keyv)
            pltpu.sync_copy(tbl_hbm.at[keyv], tv)
            pltpu.sync_copy(tv, rev_hbm.at[pl.ds(off, CH)])
            return 0
        lax.fori_loop(0, NCHUNK, fin, 0)

    return k


@functools.cache
def _sc_head():
    @functools.partial(
        pl.kernel,
        out_type=[jax.ShapeDtypeStruct((E, SDIM), f32),
                  jax.ShapeDtypeStruct((E, SDIM), f32),
                  jax.ShapeDtypeStruct((E, 8), f32),
                  jax.ShapeDtypeStruct((E, 8), f32),
                  jax.ShapeDtypeStruct((E, EDIM), f32),
                  jax.ShapeDtypeStruct((E, EDIM), f32)],
        mesh=_mesh(),
        compiler_params=_SC_PARAMS,
        scratch_types=[pltpu.VMEM((CH,), i32), pltpu.VMEM((CH,), i32),
                       pltpu.VMEM((CH,), i32),
                       pltpu.VMEM((CH, SDIM), f32), pltpu.VMEM((CH, 8), f32),
                       pltpu.VMEM((CH, EDIM), f32)],
    )
    def k(s2_hbm, cp_hbm, eext_hbm, src_hbm, dst_hbm, fwd_hbm, rev_hbm,
          s1o, s2o, cpio, cpjo, efo, ero,
          isv, idv, iwv, rows, rows4, rows32):
        base0 = _wid() * EPW

        def body(i, _):
            off = base0 + i * CH
            pltpu.sync_copy(src_hbm.at[pl.ds(off, CH)], isv)
            pltpu.sync_copy(dst_hbm.at[pl.ds(off, CH)], idv)
            pltpu.sync_copy(s2_hbm.at[idv], rows)
            pltpu.sync_copy(rows, s1o.at[pl.ds(off, CH)])
            pltpu.sync_copy(s2_hbm.at[isv], rows)
            pltpu.sync_copy(rows, s2o.at[pl.ds(off, CH)])
            pltpu.sync_copy(cp_hbm.at[idv], rows4)
            pltpu.sync_copy(rows4, cpio.at[pl.ds(off, CH)])
            pltpu.sync_copy(cp_hbm.at[isv], rows4)
            pltpu.sync_copy(rows4, cpjo.at[pl.ds(off, CH)])
            pltpu.sync_copy(fwd_hbm.at[pl.ds(off, CH)], iwv)
            pltpu.sync_copy(eext_hbm.at[iwv], rows32)
            pltpu.sync_copy(rows32, efo.at[pl.ds(off, CH)])
            pltpu.sync_copy(rev_hbm.at[pl.ds(off, CH)], iwv)
            def sel(j, _):
                rv = iwv[pl.ds(j * 16, 16)]
                iwv[pl.ds(j * 16, 16)] = jnp.where(rv < 0, E, rv)
                return 0
            lax.fori_loop(0, CH // 16, sel, 0)
            pltpu.sync_copy(eext_hbm.at[iwv], rows32)
            pltpu.sync_copy(rows32, ero.at[pl.ds(off, CH)])
            return 0

        lax.fori_loop(0, NCHUNK, body, 0)

    return k


def _silu(x):
    return x * jax.nn.sigmoid(x)


@functools.cache
def _tc_pre_node():
    def body(x_ref, t_ref, batch_ref, cnt_ref, W_atom, b_atom, W_ta, b_ta,
             W_att, b_att, W_tb, b_tb, W_btt, b_btt, W1a, W1b,
             s_ref, p_ref, q_ref, tb3_ref, icnt_ref):
        t = t_ref[...]
        ta2 = jnp.dot(t * W_ta[...] + b_ta[...], W_att[...],
                      preferred_element_type=f32) + b_att[...]
        tb2 = (t * W_tb[...] + b_tb[...]) @ W_btt[...] + b_btt[...]
        oh = (batch_ref[...] == lax.broadcasted_iota(i32, (1, B), 1)).astype(f32)
        s0 = jnp.dot(x_ref[...], W_atom[...], preferred_element_type=f32) \
            + b_atom[...] + jnp.dot(oh, ta2, preferred_element_type=f32)
        s_ref[...] = s0
        tb3_ref[...] = jnp.dot(oh, tb2, preferred_element_type=f32)
        p_ref[...] = jnp.dot(s0, W1a[...], preferred_element_type=f32)
        q_ref[...] = jnp.dot(s0, W1b[...], preferred_element_type=f32)
        cnt = cnt_ref[0, :] + cnt_ref[1, :]
        icnt_ref[...] = (1.0 / jnp.maximum(cnt, 1.0)).reshape(N, 1)

    return pl.pallas_call(
        body,
        out_shape=[jax.ShapeDtypeStruct((N, SDIM), f32),
                   jax.ShapeDtypeStruct((N, SDIM), f32),
                   jax.ShapeDtypeStruct((N, SDIM), f32),
                   jax.ShapeDtypeStruct((N, EDIM), f32),
                   jax.ShapeDtypeStruct((N, 1), f32)],
    )


BE = 1024


@functools.cache
def _tc_pre_edge():
    def body(ea_ref, tbg_ref, ps_ref, pd_ref, src_ref, dst_ref, W_bond, b_bond,
             e_ref, d_ref, a_ref, rn_ref, key_ref, rkey_ref):
        ps = ps_ref[...]
        pd = pd_ref[...]
        e_ref[...] = jnp.dot(ea_ref[...], W_bond[...], preferred_element_type=f32) \
            + b_bond[...] + tbg_ref[...]
        r = pd - ps
        rr = jnp.sum(r * r, axis=-1, keepdims=True)
        d = jnp.sqrt(jnp.clip(rr, 1e-6, None))
        d_ref[...] = d
        ns = jnp.sqrt(jnp.sum(ps * ps, axis=-1, keepdims=True))
        nd = jnp.sqrt(jnp.sum(pd * pd, axis=-1, keepdims=True))
        pns = jnp.where(ns != 0.0, ps / jnp.where(ns == 0.0, 1.0, ns), 0.0)
        pnd = jnp.where(nd != 0.0, pd / jnp.where(nd == 0.0, 1.0, nd), 0.0)
        a_ref[...] = jnp.sum(pns * pnd, axis=-1, keepdims=True)
        rn_ref[...] = (r / (1.0 + d))[:, 0:4]
        key_ref[...] = src_ref[...] * N + dst_ref[...]
        rkey_ref[...] = dst_ref[...] * N + src_ref[...]

    grid = (E // BE,)
    eb = lambda w: pl.BlockSpec((BE, w), lambda i: (i, 0))
    wb = lambda s: pl.BlockSpec(s, lambda i: (0,) * len(s))
    return pl.pallas_call(
        body,
        grid=grid,
        in_specs=[eb(NBT), eb(EDIM), eb(8), eb(8), eb(1), eb(1),
                  wb((NBT, EDIM)), wb((1, EDIM))],
        out_specs=[eb(EDIM), eb(1), eb(1), eb(4), eb(1), eb(1)],
        out_shape=[jax.ShapeDtypeStruct((E, EDIM), f32),
                   jax.ShapeDtypeStruct((E, 1), f32),
                   jax.ShapeDtypeStruct((E, 1), f32),
                   jax.ShapeDtypeStruct((E, 4), f32),
                   jax.ShapeDtypeStruct((E, 1), i32),
                   jax.ShapeDtypeStruct((E, 1), i32)],
    )


@functools.cache
def _tc_layer_node():
    def body(s_ref, v_ref, sg0_ref, sg1_ref, vp0_ref, vp1_ref, icnt_ref,
             W2, b2, W1a, W1b, s_new_ref, v_new_ref, p_ref, q_ref):
        icnt = icnt_ref[...]
        seg = (sg0_ref[...] + sg1_ref[...]) * icnt
        s_new = s_ref[...] + jnp.dot(seg, W2[...], preferred_element_type=f32) + b2[...]
        s_new_ref[...] = s_new
        v_new_ref[...] = v_ref[...] + (vp0_ref[...] + vp1_ref[...]) * icnt
        p_ref[...] = jnp.dot(s_new, W1a[...], preferred_element_type=f32)
        q_ref[...] = jnp.dot(s_new, W1b[...], preferred_element_type=f32)

    return pl.pallas_call(
        body,
        out_shape=[jax.ShapeDtypeStruct((N, SDIM), f32),
                   jax.ShapeDtypeStruct((N, V3), f32),
                   jax.ShapeDtypeStruct((N, SDIM), f32),
                   jax.ShapeDtypeStruct((N, SDIM), f32)],
    )


def _tc_edge_msg_body(first, g1_ref, g2_ref, vs_ref, e_ref, d_ref, a_ref, rn_ref,
                      W1c, wd, wa, b1, We, be_, Wg, Wg2,
                      m_ref, enew_ref, vm_ref):
    pre = g1_ref[...] + g2_ref[...] \
        + jnp.dot(e_ref[...], W1c[...], preferred_element_type=f32) \
        + d_ref[...] * wd[...] + a_ref[...] * wa[...] + b1[...]
    m = _silu(pre)
    m_ref[...] = m
    enew_ref[...] = e_ref[...] + jnp.dot(m, We[...], preferred_element_type=f32) + be_[...]
    g = jnp.dot(m, Wg[...], preferred_element_type=f32)
    rn = rn_ref[...]
    vm = jnp.concatenate([rn[:, 0:1] * g, rn[:, 1:2] * g, rn[:, 2:3] * g], axis=-1)
    if not first:
        g2g = jnp.dot(m, Wg2[...], preferred_element_type=f32)
        vm = vm + vs_ref[...] * jnp.concatenate([g2g, g2g, g2g], axis=-1)
    vm_ref[...] = vm


@functools.cache
def _tc_edge_msg(first):
    grid = (E // BE,)
    eb = lambda w: pl.BlockSpec((BE, w), lambda i: (i, 0))
    wb = lambda s: pl.BlockSpec(s, lambda i: (0,) * len(s))
    in_specs = [eb(SDIM), eb(SDIM)]
    if not first:
        in_specs.append(eb(V3))
    in_specs += [eb(EDIM), eb(1), eb(1), eb(4),
                 wb((EDIM, SDIM)), wb((1, SDIM)), wb((1, SDIM)), wb((1, SDIM)),
                 wb((SDIM, EDIM)), wb((1, EDIM)),
                 wb((SDIM, VDIM)), wb((SDIM, VDIM))]

    def body(*refs):
        if first:
            g1, g2, e, d, a, rn, W1c, wd, wa, b1, We, be_, Wg, Wg2, m, en, vm = refs
            _tc_edge_msg_body(True, g1, g2, None, e, d, a, rn,
                              W1c, wd, wa, b1, We, be_, Wg, Wg2, m, en, vm)
        else:
            g1, g2, vs, e, d, a, rn, W1c, wd, wa, b1, We, be_, Wg, Wg2, m, en, vm = refs
            _tc_edge_msg_body(False, g1, g2, vs, e, d, a, rn,
                              W1c, wd, wa, b1, We, be_, Wg, Wg2, m, en, vm)

    return pl.pallas_call(
        body,
        grid=grid,
        in_specs=in_specs,
        out_specs=[eb(SDIM), eb(EDIM), eb(V3)],
        out_shape=[jax.ShapeDtypeStruct((E, SDIM), f32),
                   jax.ShapeDtypeStruct((E, EDIM), f32),
                   jax.ShapeDtypeStruct((E, V3), f32)],
    )


@functools.cache
def _tc_head_node():
    def body(s_ref, v_ref, sg0_ref, sg1_ref, vp0_ref, vp1_ref, icnt_ref,
             pos_ref, batch_ref, W2, b2, W_sh, b_sh, W_ao, b_ao, W_co,
             W_b0a, W_bm, b_bm, b_b0,
             atoms_ref, cp_ref, s2_ref, wcomb_ref, bias0_ref):
        icnt = icnt_ref[...]
        seg = (sg0_ref[...] + sg1_ref[...]) * icnt
        s5 = s_ref[...] + jnp.dot(seg, W2[...], preferred_element_type=f32) + b2[...]
        v5 = v_ref[...] + (vp0_ref[...] + vp1_ref[...]) * icnt
        sh = _silu(jnp.dot(s5, W_sh[...], preferred_element_type=f32) + b_sh[...])
        atoms_ref[...] = jnp.dot(sh, W_ao[...], preferred_element_type=f32) + b_ao[...]
        wco = W_co[...]
        c0 = jnp.sum(v5[:, 0:VDIM] * wco, axis=-1, keepdims=True)
        c1 = jnp.sum(v5[:, VDIM:2 * VDIM] * wco, axis=-1, keepdims=True)
        c2 = jnp.sum(v5[:, 2 * VDIM:] * wco, axis=-1, keepdims=True)
        cp0 = pos_ref[...] + jnp.concatenate(
            [c0, c1, c2, jnp.zeros((N, 5), f32)], axis=-1)
        oh = (batch_ref[...] == lax.broadcasted_iota(i32, (1, B), 1)).astype(f32)
        cp0e = jnp.concatenate([cp0, jnp.ones((N, 1), f32)], axis=-1)
        sums = lax.dot_general(oh, cp0e, (((0,), (0,)), ((), ())),
                               preferred_element_type=f32)
        means = sums[:, 0:8] / jnp.maximum(sums[:, 8:9], 1.0)
        cp_ref[...] = cp0 - jnp.dot(oh, means, preferred_element_type=f32)
        s2_ref[...] = jnp.dot(sh, W_b0a[...], preferred_element_type=f32)
        wcomb_ref[...] = jnp.dot(W_bm[...], W_b0a[...], preferred_element_type=f32)
        bias0_ref[...] = jnp.dot(b_bm[...], W_b0a[...],
                                 preferred_element_type=f32) + b_b0[...]

    return pl.pallas_call(
        body,
        out_shape=[jax.ShapeDtypeStruct((N, NAF), f32),
                   jax.ShapeDtypeStruct((N, 8), f32),
                   jax.ShapeDtypeStruct((N, SDIM), f32),
                   jax.ShapeDtypeStruct((EDIM, SDIM), f32),
                   jax.ShapeDtypeStruct((1, SDIM), f32)],
    )


@functools.cache
def _tc_head_edge():
    def body(s1_ref, s2_ref, ef_ref, er_ref, cpi_ref, cpj_ref,
             wcomb, bias0, wdd, W_b1, b_b1, bonds_ref):
        diff = cpi_ref[...] - cpj_ref[...]
        dd = jnp.sum(diff * diff, axis=-1, keepdims=True)
        es = 0.5 * (ef_ref[...] + er_ref[...])
        pre = s1_ref[...] + s2_ref[...] \
            + jnp.dot(es, wcomb[...], preferred_element_type=f32) \
            + dd * wdd[...] + bias0[...]
        h = _silu(pre)
        bonds_ref[...] = jnp.dot(h, W_b1[...], preferred_element_type=f32) + b_b1[...]

    grid = (E // BE,)
    eb = lambda w: pl.BlockSpec((BE, w), lambda i: (i, 0))
    wb = lambda s: pl.BlockSpec(s, lambda i: (0,) * len(s))
    return pl.pallas_call(
        body,
        grid=grid,
        in_specs=[eb(SDIM), eb(SDIM), eb(EDIM), eb(EDIM), eb(8), eb(8),
                  wb((EDIM, SDIM)), wb((1, SDIM)), wb((1, SDIM)),
                  wb((SDIM, NBT)), wb((1, NBT))],
        out_specs=[eb(NBT)],
        out_shape=[jax.ShapeDtypeStruct((E, NBT), f32)],
    )


def kernel(x, t, pos, edge_index, edge_attr, batch, params):
    src = edge_index[0]
    dst = edge_index[1]
    batch2d = batch.reshape(N, 1)
    pos4 = jnp.concatenate([pos, jnp.zeros((N, 5), f32)], axis=-1)
    src2d = src.reshape(E, 1)
    dst2d = dst.reshape(E, 1)

    W1 = params['W1']
    W1a = [W1[l, :SDIM] for l in range(NL)]
    W1b = [W1[l, SDIM:2 * SDIM] for l in range(NL)]
    W1c = [W1[l, 2 * SDIM:2 * SDIM + EDIM] for l in range(NL)]
    wd = [W1[l, 2 * SDIM + EDIM].reshape(1, SDIM) for l in range(NL)]
    wa = [W1[l, 2 * SDIM + EDIM + 1].reshape(1, SDIM) for l in range(NL)]
    b1 = [params['b1'][l].reshape(1, SDIM) for l in range(NL)]

    def tb3_body(t_ref, batch_ref, W_tb, b_tb, W_btt, b_btt, tb3_ref):
        tb2 = (t_ref[...] * W_tb[...] + b_tb[...]) @ W_btt[...] + b_btt[...]
        oh = (batch_ref[...] == lax.broadcasted_iota(i32, (1, B), 1)).astype(f32)
        tb3_ref[...] = jnp.dot(oh, tb2, preferred_element_type=f32)

    tb3 = pl.pallas_call(
        tb3_body, out_shape=jax.ShapeDtypeStruct((N, EDIM), f32),
    )(t, batch2d, params['W_tb'], params['b_tb'], params['W_btt'], params['b_btt'])

    ones_ch = jnp.ones((CH,), f32)
    zeros_n = jnp.zeros((N,), f32)
    ps4, pd4, tbg, cnt2 = _sc_pre()(pos4, tb3, src, dst, ones_ch, zeros_n)

    s0, P, Q, _tb3_unused, inv_cnt = _tc_pre_node()(
        x, t, batch2d, cnt2, params['W_atom'], params['b_atom'],
        params['W_ta'], params['b_ta'], params['W_att'], params['b_att'],
        params['W_tb'], params['b_tb'], params['W_btt'], params['b_btt'],
        W1a[0], W1b[0])

    e0, d_e, a_e, rn4, key2d, rkey2d = _tc_pre_edge()(
        edge_attr, tbg, ps4, pd4, src2d, dst2d,
        params['W_bond'], params['b_bond'].reshape(1, EDIM))

    zs = jnp.zeros((N, SDIM), f32)
    zv = jnp.zeros((N, V3), f32)

    s, e, v = s0, e0, None
    segm = segv = None
    for l in range(NL):
        if l == 0:
            G1, G2 = _sc_gather2()(P, Q, src, dst)
            m, e, vm = _tc_edge_msg(True)(
                G1, G2, e, d_e, a_e, rn4,
                W1c[0], wd[0], wa[0], b1[0],
                params['We'][0], params['be'][0].reshape(1, EDIM),
                params['Wg'][0], params['Wg2'][0])
        else:
            s, v, P, Q = _tc_layer_node()(
                s, v if v is not None else jnp.zeros((N, V3), f32),
                segm[0], segm[1], segv[0], segv[1], inv_cnt,
                params['W2'][l - 1], params['b2'][l - 1].reshape(1, SDIM),
                W1a[l], W1b[l])
            G1, G2, Vs = _sc_gather3()(P, Q, v, src, dst)
            m, e, vm = _tc_edge_msg(False)(
                G1, G2, Vs, e, d_e, a_e, rn4,
                W1c[l], wd[l], wa[l], b1[l],
                params['We'][l], params['be'][l].reshape(1, EDIM),
                params['Wg'][l], params['Wg2'][l])
        segm, segv = _sc_scat2()(m, vm, dst, zs, zv)
        if l == 0:
            v = jnp.zeros((N, V3), f32)

    atoms_pred, cp4, s2, wcomb, bias0 = _tc_head_node()(
        s, v, segm[0], segm[1], segv[0], segv[1], inv_cnt, pos4, batch2d,
        params['W2'][NL - 1], params['b2'][NL - 1].reshape(1, SDIM),
        params['W_sh'], params['b_sh'].reshape(1, SDIM),
        params['W_ao'], params['b_ao'].reshape(1, NAF),
        params['W_co'].reshape(1, VDIM),
        params['W_b0'][:SDIM], params['W_bm'],
        params['b_bm'].reshape(1, SDIM), params['b_b0'].reshape(1, SDIM))

    key = key2d.reshape(E)
    rkey = rkey2d.reshape(E)
    ids = jnp.arange(E, dtype=i32)
    neg1 = jnp.full((16384,), -1, i32)
    fwd, rev, _tbl = _sc_sym()(key, rkey, ids, neg1)

    e_ext = jnp.concatenate([e, jnp.zeros((8, EDIM), f32)], axis=0)
    S1, S2, cpi, cpj, ef, er = _sc_head()(s2, cp4, e_ext, src, dst, fwd, rev)

    bonds_pred, = _tc_head_edge()(
        S1, S2, ef, er, cpi, cpj, wcomb, bias0,
        params['W_b0'][SDIM].reshape(1, SDIM),
        params['W_b1'], params['b_b1'].reshape(1, NBT))

    coords_pred = cp4[:, :3]
    return coords_pred, atoms_pred, bonds_pred
